# Initial kernel scaffold; baseline (speedup 1.0000x reference)
import jax
import jax.numpy as jnp
from jax.experimental import pallas as pl


def kernel(input_ids, token_types, n_lower, n_upper, n_alpha, n_spaces, n_numeric, n_special, rx_ids, ry_ids, edge_index, edge_weights, we_table, nl_table, nu_table, na_table, nsp_table, nnum_table, nspec_table, tt_table, rx_table, ry_table, W_in, b_in, W_gcn, W_self, b_gcn, W_proj, b_proj, Ws_head, bs_head, Ws_tail, bs_tail, fields_s, Ws0, Ws1, Wg_head, bg_head, Wg_tail, bg_tail, fields_g, Wg0, Wg1):
    raise NotImplementedError("write your pallas kernel here")


# speedup vs baseline: 1.1155x; 1.1155x over previous
import functools

import jax
import jax.numpy as jnp
from jax import lax
from jax.experimental import pallas as pl
from jax.experimental.pallas import tpu as pltpu
from jax.experimental.pallas import tpu_sc as plsc

N = 2048
E = 32768
VOCAB = 100000
U_TEXT = 512
U_DIST = 512
D_EPART = 12
D_GATHER = 16
D_MODEL = 128
N_FIELDS = 10
NFEAT = 10

NC = 2
NS = 16
L = 16
NW = NC * NS
NODES_PER_W = N // NW
EDGES_PER_W = E // NW
ECHUNK = 256
NECH = EDGES_PER_W // ECHUNK
ACC_ROWS_PER_TILE = N // NS

TABLE_ROWS = (VOCAB, U_TEXT, U_TEXT, U_TEXT, U_TEXT, U_TEXT, U_TEXT, 4,
              U_DIST, U_DIST)

@functools.cache
def _sc_mesh():
    return plsc.VectorSubcoreMesh(core_axis_name="c", subcore_axis_name="s",
                                  num_cores=NC, num_subcores=NS)


def _sc_embed_body(*refs):
    idx_refs = refs[0:NFEAT]
    tbl_refs = refs[NFEAT:2 * NFEAT]
    out = refs[2 * NFEAT]
    idx_v, gidx_v, rem_v, gbuf, rows_v, sem = refs[2 * NFEAT + 1:]
    c = lax.axis_index("c")
    s = lax.axis_index("s")
    base = (c * NS + s) * NODES_PER_W
    iota = lax.iota(jnp.int32, L)
    for f in range(NFEAT):
        pltpu.sync_copy(idx_refs[f].at[pl.ds(base, NODES_PER_W)], idx_v)
        ub = TABLE_ROWS[f] - 1
        for k in range(NODES_PER_W // L):
            v = idx_v[pl.ds(k * L, L)]
            v = jnp.minimum(jnp.maximum(v, 0), ub)
            gidx_v[pl.ds(k * L, L)] = v >> 3
            rem_v[pl.ds(k * L, L)] = (v & 7) * D_GATHER
        pltpu.async_copy(tbl_refs[f].at[gidx_v], gbuf, sem).wait()

        def extract(g, _):
            rem16 = rem_v[pl.ds(g * L, L)]
            for e in range(L):
                n = g * L + e
                col = jnp.full((L,), rem16[e], jnp.int32) + iota
                row = jnp.full((L,), n, jnp.int32)
                rows_v[n, :] = plsc.load_gather(gbuf, [row, col])
            return 0
        lax.fori_loop(0, NODES_PER_W // L, extract, 0)
        pltpu.sync_copy(rows_v, out.at[f, pl.ds(base, NODES_PER_W)])


@functools.cache
def _sc_embed_kernel():
    return pl.kernel(
        _sc_embed_body,
        out_type=jax.ShapeDtypeStruct((NFEAT, N, D_GATHER), jnp.float32),
        mesh=_sc_mesh(),
        scratch_types=[
            pltpu.VMEM((NODES_PER_W,), jnp.int32),
            pltpu.VMEM((NODES_PER_W,), jnp.int32),
            pltpu.VMEM((NODES_PER_W,), jnp.int32),
            pltpu.VMEM((NODES_PER_W, 8 * D_GATHER), jnp.float32),
            pltpu.VMEM((NODES_PER_W, D_GATHER), jnp.float32),
            pltpu.SemaphoreType.DMA,
        ],
        compiler_params=pltpu.CompilerParams(needs_layout_passes=False),
    )


def _sc_embed(*args):
    return _sc_embed_kernel()(*args)


def _sc_edge_body(h0, srcr, dstr, ewr, out, acc, sidx, didx, wbuf, msg, obuf,
                  sem):
    c = lax.axis_index("c")
    s = lax.axis_index("s")

    def zrow(i, _):
        for j in range(D_MODEL // L):
            obuf[i, pl.ds(j * L, L)] = jnp.zeros((L,), jnp.float32)
        return 0
    lax.fori_loop(0, ACC_ROWS_PER_TILE, zrow, 0)
    pltpu.sync_copy(obuf, acc.at[pl.ds(s * ACC_ROWS_PER_TILE,
                                       ACC_ROWS_PER_TILE)])
    plsc.subcore_barrier()

    base = (c * NS + s) * EDGES_PER_W
    for ch in range(NECH):
        off = base + ch * ECHUNK
        pltpu.sync_copy(srcr.at[pl.ds(off, ECHUNK)], sidx)
        pltpu.sync_copy(dstr.at[pl.ds(off, ECHUNK)], didx)
        pltpu.sync_copy(ewr.at[pl.ds(off, ECHUNK)], wbuf)
        pltpu.async_copy(h0.at[sidx], msg, sem).wait()

        def scale_group(g, _):
            w16 = wbuf[pl.ds(g * L, L)]
            for e in range(L):
                wv = jnp.full((L,), w16[e], jnp.float32)
                i = g * L + e
                for j in range(D_MODEL // L):
                    msg[i, pl.ds(j * L, L)] = msg[i, pl.ds(j * L, L)] * wv
            return 0
        lax.fori_loop(0, ECHUNK // L, scale_group, 0)
        pltpu.sync_copy(msg, acc.at[didx], add=True)

    plsc.subcore_barrier()
    pltpu.sync_copy(acc.at[pl.ds(s * ACC_ROWS_PER_TILE, ACC_ROWS_PER_TILE)],
                    obuf)
    pltpu.sync_copy(obuf, out.at[c, pl.ds(s * ACC_ROWS_PER_TILE,
                                          ACC_ROWS_PER_TILE)])


@functools.cache
def _sc_edges_kernel():
    return pl.kernel(
        _sc_edge_body,
        out_type=jax.ShapeDtypeStruct((NC, N, D_MODEL), jnp.float32),
        mesh=_sc_mesh(),
        scratch_types=[
            pltpu.VMEM_SHARED((N, D_MODEL), jnp.float32),
            pltpu.VMEM((ECHUNK,), jnp.int32),
            pltpu.VMEM((ECHUNK,), jnp.int32),
            pltpu.VMEM((ECHUNK,), jnp.float32),
            pltpu.VMEM((ECHUNK, D_MODEL), jnp.float32),
            pltpu.VMEM((ACC_ROWS_PER_TILE, D_MODEL), jnp.float32),
            pltpu.SemaphoreType.DMA,
        ],
    )


def _sc_edges(*args):
    return _sc_edges_kernel()(*args)


def _tc_h0_body(parts_ref, w_ref, b_ref, out_ref):
    acc = None
    for f in range(NFEAT):
        t = jnp.dot(parts_ref[f], w_ref[f],
                    preferred_element_type=jnp.float32)
        acc = t if acc is None else acc + t
    out_ref[...] = acc + b_ref[...]


def _tc_h0(parts, w3, b2):
    return pl.pallas_call(
        _tc_h0_body,
        out_shape=jax.ShapeDtypeStruct((N, D_MODEL), jnp.float32),
    )(parts, w3, b2)


def _tc_prep_body(aggp, h0, Wg, Wslf, bg, Wp, bp,
                  Wsh, bsh, Wst, bst, Ws0, Ws1,
                  Wgh, bgh, Wgt, bgt, Wg0, Wg1,
                  heads_out, t_out):
    f32 = jnp.float32
    agg = aggp[0] + aggp[1]
    pre = (jnp.dot(agg, Wg[...], preferred_element_type=f32)
           + jnp.dot(h0[...], Wslf[...], preferred_element_type=f32)
           + bg[...])
    h = jnp.maximum(pre, 0.0)
    enc = jnp.dot(h, Wp[...], preferred_element_type=f32) + bp[...]
    combos = ((Wsh, bsh, Wst, bst, Ws0, Ws1),
              (Wgh, bgh, Wgt, bgt, Wg0, Wg1))
    for a, (Wh, bh, Wt, bt, W0, W1) in enumerate(combos):
        head = jnp.dot(enc, Wh[...], preferred_element_type=f32) + bh[...]
        tail = jnp.dot(enc, Wt[...], preferred_element_type=f32) + bt[...]
        heads_out[a] = head
        t_out[a, 0] = jnp.dot(tail, W0[...], preferred_element_type=f32)
        t_out[a, 1] = jnp.dot(tail, W1[...], preferred_element_type=f32)


def _tc_prep(aggp, h0, *ws):
    return pl.pallas_call(
        _tc_prep_body,
        out_shape=[
            jax.ShapeDtypeStruct((2, N, D_MODEL), jnp.float32),
            jax.ShapeDtypeStruct((2, 2, N, D_MODEL), jnp.float32),
        ],
    )(aggp, h0, *ws)


def _tc_score_body(h_ref, t_ref, o_ref):
    o_ref[0, 0] = lax.dot_general(
        h_ref[0], t_ref[0, 0],
        dimension_numbers=(((1,), (1,)), ((), ())),
        preferred_element_type=jnp.float32)


def _tc_scores(hcat, t):
    nrows = N + N_FIELDS
    return pl.pallas_call(
        _tc_score_body,
        grid=(2, 2),
        in_specs=[
            pl.BlockSpec((1, nrows, D_MODEL), lambda a, b: (a, 0, 0)),
            pl.BlockSpec((1, 1, N, D_MODEL), lambda a, b: (a, b, 0, 0)),
        ],
        out_specs=pl.BlockSpec((1, 1, nrows, N), lambda a, b: (a, b, 0, 0)),
        out_shape=jax.ShapeDtypeStruct((2, 2, nrows, N), jnp.float32),
    )(hcat, t)


def kernel(input_ids, token_types, n_lower, n_upper, n_alpha, n_spaces,
           n_numeric, n_special, rx_ids, ry_ids, edge_index, edge_weights,
           we_table, nl_table, nu_table, na_table, nsp_table, nnum_table,
           nspec_table, tt_table, rx_table, ry_table, W_in, b_in, W_gcn,
           W_self, b_gcn, W_proj, b_proj, Ws_head, bs_head, Ws_tail, bs_tail,
           fields_s, Ws0, Ws1, Wg_head, bg_head, Wg_tail, bg_tail, fields_g,
           Wg0, Wg1):
    i32 = jnp.int32
    idxs = [x.astype(i32) for x in
            (input_ids, n_lower, n_upper, n_alpha, n_spaces, n_numeric,
             n_special, token_types, rx_ids, ry_ids)]
    def grp(t):
        r = (-t.shape[0]) % 8
        t = jnp.pad(t, ((0, r), (0, D_GATHER - t.shape[1])))
        return t.reshape(t.shape[0] // 8, 8 * D_GATHER)
    tables = tuple(grp(t) for t in
                   (we_table, nl_table, nu_table, na_table, nsp_table,
                    nnum_table, nspec_table, tt_table, rx_table, ry_table))
    parts = _sc_embed(*idxs, *tables)

    w3 = jnp.pad(W_in.reshape(NFEAT, D_EPART, D_MODEL),
                 ((0, 0), (0, D_GATHER - D_EPART), (0, 0)))
    h0 = _tc_h0(parts, w3, b_in.reshape(1, D_MODEL))

    src = edge_index[0].astype(i32)
    dst = edge_index[1].astype(i32)
    aggp = _sc_edges(h0, src, dst, edge_weights)

    heads, t = _tc_prep(
        aggp, h0, W_gcn, W_self, b_gcn.reshape(1, D_MODEL), W_proj,
        b_proj.reshape(1, D_MODEL),
        Ws_head, bs_head.reshape(1, D_MODEL), Ws_tail,
        bs_tail.reshape(1, D_MODEL), Ws0, Ws1,
        Wg_head, bg_head.reshape(1, D_MODEL), Wg_tail,
        bg_tail.reshape(1, D_MODEL), Wg0, Wg1)

    fields = jnp.stack([fields_s, fields_g], axis=0)
    hcat = jnp.concatenate([fields, heads], axis=1)
    return _tc_scores(hcat, t)


# speedup vs baseline: 1.5650x; 1.4029x over previous
import functools

import jax
import jax.numpy as jnp
from jax import lax
from jax.experimental import pallas as pl
from jax.experimental.pallas import tpu as pltpu
from jax.experimental.pallas import tpu_sc as plsc

N = 2048
E = 32768
VOCAB = 100000
U_TEXT = 512
U_DIST = 512
D_EPART = 12
D_GATHER = 16
D_MODEL = 128
N_FIELDS = 10
NFEAT = 10

NC = 2
NS = 16
L = 16
NW = NC * NS
NODES_PER_W = N // NW
EDGES_PER_W = E // NW
ECHUNK = 256
NECH = EDGES_PER_W // ECHUNK
ACC_ROWS_PER_TILE = N // NS

TABLE_ROWS = (VOCAB, U_TEXT, U_TEXT, U_TEXT, U_TEXT, U_TEXT, U_TEXT, 4,
              U_DIST, U_DIST)

@functools.cache
def _sc_mesh():
    return plsc.VectorSubcoreMesh(core_axis_name="c", subcore_axis_name="s",
                                  num_cores=NC, num_subcores=NS)


def _sc_embed_body(*refs):
    idx_refs = refs[0:NFEAT]
    tbl_refs = refs[NFEAT:2 * NFEAT]
    out = refs[2 * NFEAT]
    ibuf, rows3, isem, dsem, osem = refs[2 * NFEAT + 1:]
    c = lax.axis_index("c")
    s = lax.axis_index("s")
    base = (c * NS + s) * NODES_PER_W

    idescs = [pltpu.async_copy(idx_refs[f].at[pl.ds(base, NODES_PER_W)],
                               ibuf.at[f], isem) for f in range(NFEAT)]
    for d in idescs:
        d.wait()

    for f in range(NFEAT):
        ub = TABLE_ROWS[f] - 1

        def fire(g, _, f=f, ub=ub):
            v = ibuf[f, pl.ds(g * L, L)]
            v = jnp.minimum(jnp.maximum(v, 0), ub)
            for e in range(L):
                n = g * L + e
                pltpu.async_copy(tbl_refs[f].at[pl.ds(v[e], 1)],
                                 rows3.at[f, pl.ds(n, 1)], dsem)
            return 0
        lax.fori_loop(0, NODES_PER_W // L, fire, 0)

    pltpu.make_async_copy(out.at[:, pl.ds(base, NODES_PER_W), :], rows3,
                          dsem).wait()
    pltpu.async_copy(rows3, out.at[:, pl.ds(base, NODES_PER_W), :],
                     osem).wait()


@functools.cache
def _sc_embed_kernel():
    return pl.kernel(
        _sc_embed_body,
        out_type=jax.ShapeDtypeStruct((NFEAT, N, D_EPART), jnp.float32),
        mesh=_sc_mesh(),
        scratch_types=[
            pltpu.VMEM((NFEAT, NODES_PER_W), jnp.int32),
            pltpu.VMEM((NFEAT, NODES_PER_W, D_EPART), jnp.float32),
            pltpu.SemaphoreType.DMA,
            pltpu.SemaphoreType.DMA,
            pltpu.SemaphoreType.DMA,
        ],
        compiler_params=pltpu.CompilerParams(needs_layout_passes=False),
    )


def _sc_embed(*args):
    return _sc_embed_kernel()(*args)


def _sc_edge_body(h0, srcr, dstr, ewr, out, acc, sidx, didx, wbuf, msg, obuf,
                  sem):
    c = lax.axis_index("c")
    s = lax.axis_index("s")

    def zrow(i, _):
        for j in range(D_MODEL // L):
            obuf[i, pl.ds(j * L, L)] = jnp.zeros((L,), jnp.float32)
        return 0
    lax.fori_loop(0, ACC_ROWS_PER_TILE, zrow, 0)
    pltpu.sync_copy(obuf, acc.at[pl.ds(s * ACC_ROWS_PER_TILE,
                                       ACC_ROWS_PER_TILE)])
    plsc.subcore_barrier()

    base = (c * NS + s) * EDGES_PER_W
    for ch in range(NECH):
        off = base + ch * ECHUNK
        pltpu.sync_copy(srcr.at[pl.ds(off, ECHUNK)], sidx)
        pltpu.sync_copy(dstr.at[pl.ds(off, ECHUNK)], didx)
        pltpu.sync_copy(ewr.at[pl.ds(off, ECHUNK)], wbuf)
        pltpu.async_copy(h0.at[sidx], msg, sem).wait()

        def scale_group(g, _):
            w16 = wbuf[pl.ds(g * L, L)]
            for e in range(L):
                wv = jnp.full((L,), w16[e], jnp.float32)
                i = g * L + e
                for j in range(D_MODEL // L):
                    msg[i, pl.ds(j * L, L)] = msg[i, pl.ds(j * L, L)] * wv
            return 0
        lax.fori_loop(0, ECHUNK // L, scale_group, 0)
        pltpu.sync_copy(msg, acc.at[didx], add=True)

    plsc.subcore_barrier()
    pltpu.sync_copy(acc.at[pl.ds(s * ACC_ROWS_PER_TILE, ACC_ROWS_PER_TILE)],
                    obuf)
    pltpu.sync_copy(obuf, out.at[c, pl.ds(s * ACC_ROWS_PER_TILE,
                                          ACC_ROWS_PER_TILE)])


@functools.cache
def _sc_edges_kernel():
    return pl.kernel(
        _sc_edge_body,
        out_type=jax.ShapeDtypeStruct((NC, N, D_MODEL), jnp.float32),
        mesh=_sc_mesh(),
        scratch_types=[
            pltpu.VMEM_SHARED((N, D_MODEL), jnp.float32),
            pltpu.VMEM((ECHUNK,), jnp.int32),
            pltpu.VMEM((ECHUNK,), jnp.int32),
            pltpu.VMEM((ECHUNK,), jnp.float32),
            pltpu.VMEM((ECHUNK, D_MODEL), jnp.float32),
            pltpu.VMEM((ACC_ROWS_PER_TILE, D_MODEL), jnp.float32),
            pltpu.SemaphoreType.DMA,
        ],
    )


def _sc_edges(*args):
    return _sc_edges_kernel()(*args)


def _tc_h0_body(parts_ref, w_ref, b_ref, out_ref):
    acc = None
    for f in range(NFEAT):
        t = jnp.dot(parts_ref[f], w_ref[f],
                    preferred_element_type=jnp.float32)
        acc = t if acc is None else acc + t
    out_ref[...] = acc + b_ref[...]


def _tc_h0(parts, w3, b2):
    return pl.pallas_call(
        _tc_h0_body,
        out_shape=jax.ShapeDtypeStruct((N, D_MODEL), jnp.float32),
    )(parts, w3, b2)


def _tc_prep_body(aggp, h0, Wg, Wslf, bg, Wp, bp,
                  Wsh, bsh, Wst, bst, Ws0, Ws1,
                  Wgh, bgh, Wgt, bgt, Wg0, Wg1,
                  heads_out, t_out):
    f32 = jnp.float32
    agg = aggp[0] + aggp[1]
    pre = (jnp.dot(agg, Wg[...], preferred_element_type=f32)
           + jnp.dot(h0[...], Wslf[...], preferred_element_type=f32)
           + bg[...])
    h = jnp.maximum(pre, 0.0)
    enc = jnp.dot(h, Wp[...], preferred_element_type=f32) + bp[...]
    combos = ((Wsh, bsh, Wst, bst, Ws0, Ws1),
              (Wgh, bgh, Wgt, bgt, Wg0, Wg1))
    for a, (Wh, bh, Wt, bt, W0, W1) in enumerate(combos):
        head = jnp.dot(enc, Wh[...], preferred_element_type=f32) + bh[...]
        tail = jnp.dot(enc, Wt[...], preferred_element_type=f32) + bt[...]
        heads_out[a] = head
        t_out[a, 0] = jnp.dot(tail, W0[...], preferred_element_type=f32)
        t_out[a, 1] = jnp.dot(tail, W1[...], preferred_element_type=f32)


def _tc_prep(aggp, h0, *ws):
    return pl.pallas_call(
        _tc_prep_body,
        out_shape=[
            jax.ShapeDtypeStruct((2, N, D_MODEL), jnp.float32),
            jax.ShapeDtypeStruct((2, 2, N, D_MODEL), jnp.float32),
        ],
    )(aggp, h0, *ws)


def _tc_score_body(h_ref, t_ref, o_ref):
    o_ref[0, 0] = lax.dot_general(
        h_ref[0], t_ref[0, 0],
        dimension_numbers=(((1,), (1,)), ((), ())),
        preferred_element_type=jnp.float32)


def _tc_scores(hcat, t):
    nrows = N + N_FIELDS
    return pl.pallas_call(
        _tc_score_body,
        grid=(2, 2),
        in_specs=[
            pl.BlockSpec((1, nrows, D_MODEL), lambda a, b: (a, 0, 0)),
            pl.BlockSpec((1, 1, N, D_MODEL), lambda a, b: (a, b, 0, 0)),
        ],
        out_specs=pl.BlockSpec((1, 1, nrows, N), lambda a, b: (a, b, 0, 0)),
        out_shape=jax.ShapeDtypeStruct((2, 2, nrows, N), jnp.float32),
    )(hcat, t)


def kernel(input_ids, token_types, n_lower, n_upper, n_alpha, n_spaces,
           n_numeric, n_special, rx_ids, ry_ids, edge_index, edge_weights,
           we_table, nl_table, nu_table, na_table, nsp_table, nnum_table,
           nspec_table, tt_table, rx_table, ry_table, W_in, b_in, W_gcn,
           W_self, b_gcn, W_proj, b_proj, Ws_head, bs_head, Ws_tail, bs_tail,
           fields_s, Ws0, Ws1, Wg_head, bg_head, Wg_tail, bg_tail, fields_g,
           Wg0, Wg1):
    i32 = jnp.int32
    idxs = [x.astype(i32) for x in
            (input_ids, n_lower, n_upper, n_alpha, n_spaces, n_numeric,
             n_special, token_types, rx_ids, ry_ids)]
    tables = (we_table, nl_table, nu_table, na_table, nsp_table, nnum_table,
              nspec_table, tt_table, rx_table, ry_table)
    parts = _sc_embed(*idxs, *tables)

    w3 = W_in.reshape(NFEAT, D_EPART, D_MODEL)
    h0 = _tc_h0(parts, w3, b_in.reshape(1, D_MODEL))

    src = edge_index[0].astype(i32)
    dst = edge_index[1].astype(i32)
    aggp = _sc_edges(h0, src, dst, edge_weights)

    heads, t = _tc_prep(
        aggp, h0, W_gcn, W_self, b_gcn.reshape(1, D_MODEL), W_proj,
        b_proj.reshape(1, D_MODEL),
        Ws_head, bs_head.reshape(1, D_MODEL), Ws_tail,
        bs_tail.reshape(1, D_MODEL), Ws0, Ws1,
        Wg_head, bg_head.reshape(1, D_MODEL), Wg_tail,
        bg_tail.reshape(1, D_MODEL), Wg0, Wg1)

    fields = jnp.stack([fields_s, fields_g], axis=0)
    hcat = jnp.concatenate([fields, heads], axis=1)
    return _tc_scores(hcat, t)


# speedup vs baseline: 2.4742x; 1.5809x over previous
import functools

import jax
import jax.numpy as jnp
from jax import lax
from jax.experimental import pallas as pl
from jax.experimental.pallas import tpu as pltpu
from jax.experimental.pallas import tpu_sc as plsc

N = 2048
E = 32768
VOCAB = 100000
U_TEXT = 512
U_DIST = 512
D_EPART = 12
D_GATHER = 16
D_MODEL = 128
N_FIELDS = 10
NFEAT = 10

NC = 2
NS = 16
L = 16
NW = NC * NS
NODES_PER_W = N // NW
EDGES_PER_W = E // NW
ECHUNK = 256
NECH = EDGES_PER_W // ECHUNK
ACC_ROWS_PER_TILE = N // NS

TABLE_ROWS = (VOCAB, U_TEXT, U_TEXT, U_TEXT, U_TEXT, U_TEXT, U_TEXT, 4,
              U_DIST, U_DIST)

@functools.cache
def _sc_mesh():
    return plsc.VectorSubcoreMesh(core_axis_name="c", subcore_axis_name="s",
                                  num_cores=NC, num_subcores=NS)


def _sc_embed_body(*refs):
    idx_refs = refs[0:NFEAT]
    tbl_refs = refs[NFEAT:2 * NFEAT]
    out = refs[2 * NFEAT]
    ibuf, rows3, isem, dsem, osem = refs[2 * NFEAT + 1:]
    c = lax.axis_index("c")
    s = lax.axis_index("s")
    base = (c * NS + s) * NODES_PER_W

    idescs = [pltpu.async_copy(idx_refs[f].at[pl.ds(base, NODES_PER_W)],
                               ibuf.at[f], isem) for f in range(NFEAT)]
    for d in idescs:
        d.wait()

    for f in range(NFEAT):
        ub = TABLE_ROWS[f] - 1

        def fire(g, _, f=f, ub=ub):
            v = ibuf[f, pl.ds(g * L, L)]
            v = jnp.minimum(jnp.maximum(v, 0), ub)
            for e in range(L):
                n = g * L + e
                pltpu.async_copy(tbl_refs[f].at[pl.ds(v[e], 1)],
                                 rows3.at[f, pl.ds(n, 1)], dsem)
            return 0
        lax.fori_loop(0, NODES_PER_W // L, fire, 0)

    pltpu.make_async_copy(out.at[:, pl.ds(base, NODES_PER_W), :], rows3,
                          dsem).wait()
    pltpu.async_copy(rows3, out.at[:, pl.ds(base, NODES_PER_W), :],
                     osem).wait()


@functools.cache
def _sc_embed_kernel():
    return pl.kernel(
        _sc_embed_body,
        out_type=jax.ShapeDtypeStruct((NFEAT, N, D_EPART), jnp.float32),
        mesh=_sc_mesh(),
        scratch_types=[
            pltpu.VMEM((NFEAT, NODES_PER_W), jnp.int32),
            pltpu.VMEM((NFEAT, NODES_PER_W, D_EPART), jnp.float32),
            pltpu.SemaphoreType.DMA,
            pltpu.SemaphoreType.DMA,
            pltpu.SemaphoreType.DMA,
        ],
        compiler_params=pltpu.CompilerParams(needs_layout_passes=False),
    )


def _sc_embed(*args):
    return _sc_embed_kernel()(*args)


def _sc_edge_body(h0, srcr, dstr, ewr, out, acc, sidx, didx, wbuf, msg, obuf,
                  sem):
    c = lax.axis_index("c")
    s = lax.axis_index("s")

    def zrow(i, _):
        for j in range(D_MODEL // L):
            obuf[i, pl.ds(j * L, L)] = jnp.zeros((L,), jnp.float32)
        return 0
    lax.fori_loop(0, ACC_ROWS_PER_TILE, zrow, 0)
    pltpu.sync_copy(obuf, acc.at[pl.ds(s * ACC_ROWS_PER_TILE,
                                       ACC_ROWS_PER_TILE)])
    plsc.subcore_barrier()

    base = (c * NS + s) * EDGES_PER_W
    for ch in range(NECH):
        off = base + ch * ECHUNK
        pltpu.sync_copy(srcr.at[pl.ds(off, ECHUNK)], sidx)
        pltpu.sync_copy(dstr.at[pl.ds(off, ECHUNK)], didx)
        pltpu.sync_copy(ewr.at[pl.ds(off, ECHUNK)], wbuf)
        pltpu.async_copy(h0.at[sidx], msg, sem).wait()

        def scale_group(g, _):
            w16 = wbuf[pl.ds(g * L, L)]
            for e in range(L):
                wv = jnp.full((L,), w16[e], jnp.float32)
                i = g * L + e
                for j in range(D_MODEL // L):
                    msg[i, pl.ds(j * L, L)] = msg[i, pl.ds(j * L, L)] * wv
            return 0
        lax.fori_loop(0, ECHUNK // L, scale_group, 0)
        pltpu.sync_copy(msg, acc.at[didx], add=True)

    plsc.subcore_barrier()
    pltpu.sync_copy(acc.at[pl.ds(s * ACC_ROWS_PER_TILE, ACC_ROWS_PER_TILE)],
                    obuf)
    pltpu.sync_copy(obuf, out.at[c, pl.ds(s * ACC_ROWS_PER_TILE,
                                          ACC_ROWS_PER_TILE)])


@functools.cache
def _sc_edges_kernel():
    return pl.kernel(
        _sc_edge_body,
        out_type=jax.ShapeDtypeStruct((NC, N, D_MODEL), jnp.float32),
        mesh=_sc_mesh(),
        scratch_types=[
            pltpu.VMEM_SHARED((N, D_MODEL), jnp.float32),
            pltpu.VMEM((ECHUNK,), jnp.int32),
            pltpu.VMEM((ECHUNK,), jnp.int32),
            pltpu.VMEM((ECHUNK,), jnp.float32),
            pltpu.VMEM((ECHUNK, D_MODEL), jnp.float32),
            pltpu.VMEM((ACC_ROWS_PER_TILE, D_MODEL), jnp.float32),
            pltpu.SemaphoreType.DMA,
        ],
    )


def _sc_edges(*args):
    return _sc_edges_kernel()(*args)


def _tc_h0_body(parts_ref, w_ref, b_ref, out_ref):
    acc = None
    for f in range(NFEAT):
        t = jnp.dot(parts_ref[f], w_ref[f],
                    preferred_element_type=jnp.float32)
        acc = t if acc is None else acc + t
    out_ref[...] = acc + b_ref[...]


def _tc_h0(parts, w3, b2):
    return pl.pallas_call(
        _tc_h0_body,
        out_shape=jax.ShapeDtypeStruct((N, D_MODEL), jnp.float32),
    )(parts, w3, b2)


def _tc_prep_body(aggp, h0, Wg, Wslf, bg, Wp, bp,
                  Wsh, bsh, Wst, bst, Ws0, Ws1,
                  Wgh, bgh, Wgt, bgt, Wg0, Wg1,
                  heads_out, t_out):
    f32 = jnp.float32
    agg = aggp[0] + aggp[1]
    pre = (jnp.dot(agg, Wg[...], preferred_element_type=f32)
           + jnp.dot(h0[...], Wslf[...], preferred_element_type=f32)
           + bg[...])
    h = jnp.maximum(pre, 0.0)
    enc = jnp.dot(h, Wp[...], preferred_element_type=f32) + bp[...]
    combos = ((Wsh, bsh, Wst, bst, Ws0, Ws1),
              (Wgh, bgh, Wgt, bgt, Wg0, Wg1))
    for a, (Wh, bh, Wt, bt, W0, W1) in enumerate(combos):
        head = jnp.dot(enc, Wh[...], preferred_element_type=f32) + bh[...]
        tail = jnp.dot(enc, Wt[...], preferred_element_type=f32) + bt[...]
        heads_out[a] = head
        t_out[a, 0] = jnp.dot(tail, W0[...], preferred_element_type=f32)
        t_out[a, 1] = jnp.dot(tail, W1[...], preferred_element_type=f32)


def _tc_prep(aggp, h0, *ws):
    return pl.pallas_call(
        _tc_prep_body,
        out_shape=[
            jax.ShapeDtypeStruct((2, N, D_MODEL), jnp.float32),
            jax.ShapeDtypeStruct((2, 2, N, D_MODEL), jnp.float32),
        ],
    )(aggp, h0, *ws)


_RB = 256


def _tc_score_body(h_ref, t_ref, o_ref):
    h = h_ref[0]
    s0 = lax.dot_general(h, t_ref[0, 0],
                         dimension_numbers=(((1,), (1,)), ((), ())),
                         preferred_element_type=jnp.float32)
    s1 = lax.dot_general(h, t_ref[0, 1],
                         dimension_numbers=(((1,), (1,)), ((), ())),
                         preferred_element_type=jnp.float32)
    x = jnp.stack([s0.reshape(_RB, N // 128, 128),
                   s1.reshape(_RB, N // 128, 128)], axis=2)
    o_ref[0] = x.reshape(_RB, 2 * (N // 128), 128)


def _tc_scores(hcat, t):
    nrows = N + N_FIELDS
    nrt = (nrows + _RB - 1) // _RB
    out4 = pl.pallas_call(
        _tc_score_body,
        grid=(2, nrt),
        in_specs=[
            pl.BlockSpec((1, _RB, D_MODEL), lambda a, r: (a, r, 0)),
            pl.BlockSpec((1, 2, N, D_MODEL), lambda a, r: (a, 0, 0, 0)),
        ],
        out_specs=pl.BlockSpec((1, _RB, 2 * (N // 128), 128),
                               lambda a, r: (a, r, 0, 0)),
        out_shape=jax.ShapeDtypeStruct((2, nrows, 2 * (N // 128), 128),
                                       jnp.float32),
    )(hcat, t)
    res = out4.reshape(2, nrows, N // 128, 2, 128)
    res = res.transpose(0, 3, 1, 2, 4)
    return res.reshape(2, 2, nrows, N)


def kernel(input_ids, token_types, n_lower, n_upper, n_alpha, n_spaces,
           n_numeric, n_special, rx_ids, ry_ids, edge_index, edge_weights,
           we_table, nl_table, nu_table, na_table, nsp_table, nnum_table,
           nspec_table, tt_table, rx_table, ry_table, W_in, b_in, W_gcn,
           W_self, b_gcn, W_proj, b_proj, Ws_head, bs_head, Ws_tail, bs_tail,
           fields_s, Ws0, Ws1, Wg_head, bg_head, Wg_tail, bg_tail, fields_g,
           Wg0, Wg1):
    i32 = jnp.int32
    idxs = [x.astype(i32) for x in
            (input_ids, n_lower, n_upper, n_alpha, n_spaces, n_numeric,
             n_special, token_types, rx_ids, ry_ids)]
    tables = (we_table, nl_table, nu_table, na_table, nsp_table, nnum_table,
              nspec_table, tt_table, rx_table, ry_table)
    parts = _sc_embed(*idxs, *tables)

    w3 = W_in.reshape(NFEAT, D_EPART, D_MODEL)
    h0 = _tc_h0(parts, w3, b_in.reshape(1, D_MODEL))

    src = edge_index[0].astype(i32)
    dst = edge_index[1].astype(i32)
    aggp = _sc_edges(h0, src, dst, edge_weights)

    heads, t = _tc_prep(
        aggp, h0, W_gcn, W_self, b_gcn.reshape(1, D_MODEL), W_proj,
        b_proj.reshape(1, D_MODEL),
        Ws_head, bs_head.reshape(1, D_MODEL), Ws_tail,
        bs_tail.reshape(1, D_MODEL), Ws0, Ws1,
        Wg_head, bg_head.reshape(1, D_MODEL), Wg_tail,
        bg_tail.reshape(1, D_MODEL), Wg0, Wg1)

    fields = jnp.stack([fields_s, fields_g], axis=0)
    hcat = jnp.concatenate([fields, heads], axis=1)
    return _tc_scores(hcat, t)


# speedup vs baseline: 2.9848x; 1.2064x over previous
import functools

import jax
import jax.numpy as jnp
from jax import lax
from jax.experimental import pallas as pl
from jax.experimental.pallas import tpu as pltpu
from jax.experimental.pallas import tpu_sc as plsc

N = 2048
E = 32768
VOCAB = 100000
U_TEXT = 512
U_DIST = 512
D_EPART = 12
D_GATHER = 16
D_MODEL = 128
N_FIELDS = 10
NFEAT = 10

NC = 2
NS = 16
L = 16
NW = NC * NS
NODES_PER_W = N // NW
EDGES_PER_W = E // NW
ECHUNK = 256
NECH = EDGES_PER_W // ECHUNK
ACC_ROWS_PER_TILE = N // NS

TABLE_ROWS = (VOCAB, U_TEXT, U_TEXT, U_TEXT, U_TEXT, U_TEXT, U_TEXT, 4,
              U_DIST, U_DIST)

@functools.cache
def _sc_mesh():
    return plsc.VectorSubcoreMesh(core_axis_name="c", subcore_axis_name="s",
                                  num_cores=NC, num_subcores=NS)


def _sc_embed_body(*refs):
    idx_refs = refs[0:NFEAT]
    tbl_refs = refs[NFEAT:2 * NFEAT]
    out = refs[2 * NFEAT]
    (ibuf, tbuf, ttbuf, wecols, pbuf,
     isem, tsem, dsem, osem) = refs[2 * NFEAT + 1:]
    c = lax.axis_index("c")
    s = lax.axis_index("s")
    base = (c * NS + s) * NODES_PER_W
    iota = lax.iota(jnp.int32, L)

    idescs = [pltpu.async_copy(idx_refs[f].at[pl.ds(base, NODES_PER_W)],
                               ibuf.at[f], isem) for f in range(NFEAT)]
    tdescs = [pltpu.async_copy(tbl_refs[k + 1],
                               ttbuf if k == 6 else tbuf.at[k if k < 6 else k - 1],
                               tsem)
              for k in range(NFEAT - 1)]
    for d in idescs:
        d.wait()

    wedescs = []
    for g in range(NODES_PER_W // L):
        v = ibuf[0, pl.ds(g * L, L)]
        v = jnp.minimum(jnp.maximum(v, 0), VOCAB - 1)
        for e in range(L):
            n = g * L + e
            wedescs.append(pltpu.async_copy(
                tbl_refs[0].at[:, pl.ds((v[e] >> 3) * 8, 8)],
                wecols.at[n], dsem))

    for d in tdescs:
        d.wait()

    for f in range(1, NFEAT):
        ub = TABLE_ROWS[f] - 1
        tt = f == 7

        def gath(g, _, f=f, ub=ub, tt=tt):
            idx16 = ibuf[f, pl.ds(g * L, L)]
            idx16 = jnp.minimum(jnp.maximum(idx16, 0), ub)
            kf = jnp.full((L,), f - 1 if f < 7 else f - 2, jnp.int32)
            for j in range(D_EPART):
                jf = jnp.full((L,), j, jnp.int32)
                if tt:
                    vals = plsc.load_gather(ttbuf, [jf, idx16])
                else:
                    vals = plsc.load_gather(tbuf, [kf, jf, idx16])
                pbuf[f, j, pl.ds(g * L, L)] = vals
            return 0
        lax.fori_loop(0, NODES_PER_W // L, gath, 0)

    for d in wedescs:
        d.wait()

    def we_extract(g, _):
        v = ibuf[0, pl.ds(g * L, L)]
        v = jnp.minimum(jnp.maximum(v, 0), VOCAB - 1)
        cv = v & 7
        nv = jnp.full((L,), g * L, jnp.int32) + iota
        for j in range(D_EPART):
            jf = jnp.full((L,), j, jnp.int32)
            pbuf[0, j, pl.ds(g * L, L)] = plsc.load_gather(
                wecols, [nv, jf, cv])
        return 0
    lax.fori_loop(0, NODES_PER_W // L, we_extract, 0)

    pltpu.async_copy(pbuf, out.at[:, :, pl.ds(base, NODES_PER_W)],
                     osem).wait()


@functools.cache
def _sc_embed_kernel():
    return pl.kernel(
        _sc_embed_body,
        out_type=jax.ShapeDtypeStruct((NFEAT, D_EPART, N), jnp.float32),
        mesh=_sc_mesh(),
        scratch_types=[
            pltpu.VMEM((NFEAT, NODES_PER_W), jnp.int32),
            pltpu.VMEM((NFEAT - 2, D_EPART, U_TEXT), jnp.float32),
            pltpu.VMEM((D_EPART, 4), jnp.float32),
            pltpu.VMEM((NODES_PER_W, D_EPART, 8), jnp.float32),
            pltpu.VMEM((NFEAT, D_EPART, NODES_PER_W), jnp.float32),
            pltpu.SemaphoreType.DMA,
            pltpu.SemaphoreType.DMA,
            pltpu.SemaphoreType.DMA,
            pltpu.SemaphoreType.DMA,
        ],
        compiler_params=pltpu.CompilerParams(needs_layout_passes=False,
                                             use_tc_tiling_on_sc=False),
    )


def _sc_embed(*args):
    return _sc_embed_kernel()(*args)


def _sc_edge_body(h0, srcr, dstr, ewr, out, acc, sidx, didx, wbuf, msg, obuf,
                  sem):
    c = lax.axis_index("c")
    s = lax.axis_index("s")

    def zrow(i, _):
        for j in range(D_MODEL // L):
            obuf[i, pl.ds(j * L, L)] = jnp.zeros((L,), jnp.float32)
        return 0
    lax.fori_loop(0, ACC_ROWS_PER_TILE, zrow, 0)
    pltpu.sync_copy(obuf, acc.at[pl.ds(s * ACC_ROWS_PER_TILE,
                                       ACC_ROWS_PER_TILE)])
    plsc.subcore_barrier()

    base = (c * NS + s) * EDGES_PER_W
    for ch in range(NECH):
        off = base + ch * ECHUNK
        pltpu.sync_copy(srcr.at[pl.ds(off, ECHUNK)], sidx)
        pltpu.sync_copy(dstr.at[pl.ds(off, ECHUNK)], didx)
        pltpu.sync_copy(ewr.at[pl.ds(off, ECHUNK)], wbuf)
        pltpu.async_copy(h0.at[sidx], msg, sem).wait()

        def scale_group(g, _):
            w16 = wbuf[pl.ds(g * L, L)]
            for e in range(L):
                wv = jnp.full((L,), w16[e], jnp.float32)
                i = g * L + e
                for j in range(D_MODEL // L):
                    msg[i, pl.ds(j * L, L)] = msg[i, pl.ds(j * L, L)] * wv
            return 0
        lax.fori_loop(0, ECHUNK // L, scale_group, 0)
        pltpu.sync_copy(msg, acc.at[didx], add=True)

    plsc.subcore_barrier()
    pltpu.sync_copy(acc.at[pl.ds(s * ACC_ROWS_PER_TILE, ACC_ROWS_PER_TILE)],
                    obuf)
    pltpu.sync_copy(obuf, out.at[c, pl.ds(s * ACC_ROWS_PER_TILE,
                                          ACC_ROWS_PER_TILE)])


@functools.cache
def _sc_edges_kernel():
    return pl.kernel(
        _sc_edge_body,
        out_type=jax.ShapeDtypeStruct((NC, N, D_MODEL), jnp.float32),
        mesh=_sc_mesh(),
        scratch_types=[
            pltpu.VMEM_SHARED((N, D_MODEL), jnp.float32),
            pltpu.VMEM((ECHUNK,), jnp.int32),
            pltpu.VMEM((ECHUNK,), jnp.int32),
            pltpu.VMEM((ECHUNK,), jnp.float32),
            pltpu.VMEM((ECHUNK, D_MODEL), jnp.float32),
            pltpu.VMEM((ACC_ROWS_PER_TILE, D_MODEL), jnp.float32),
            pltpu.SemaphoreType.DMA,
        ],
    )


def _sc_edges(*args):
    return _sc_edges_kernel()(*args)


def _tc_h0_body(parts_ref, w_ref, b_ref, out_ref):
    acc = None
    for f in range(NFEAT):
        t = lax.dot_general(parts_ref[f], w_ref[f],
                            dimension_numbers=(((0,), (0,)), ((), ())),
                            preferred_element_type=jnp.float32)
        acc = t if acc is None else acc + t
    out_ref[...] = acc + b_ref[...]


def _tc_h0(parts, w3, b2):
    return pl.pallas_call(
        _tc_h0_body,
        out_shape=jax.ShapeDtypeStruct((N, D_MODEL), jnp.float32),
    )(parts, w3, b2)


def _tc_prep_body(aggp, h0, Wg, Wslf, bg, Wp, bp,
                  Wsh, bsh, Wst, bst, Ws0, Ws1,
                  Wgh, bgh, Wgt, bgt, Wg0, Wg1,
                  heads_out, t_out):
    f32 = jnp.float32
    agg = aggp[0] + aggp[1]
    pre = (jnp.dot(agg, Wg[...], preferred_element_type=f32)
           + jnp.dot(h0[...], Wslf[...], preferred_element_type=f32)
           + bg[...])
    h = jnp.maximum(pre, 0.0)
    enc = jnp.dot(h, Wp[...], preferred_element_type=f32) + bp[...]
    combos = ((Wsh, bsh, Wst, bst, Ws0, Ws1),
              (Wgh, bgh, Wgt, bgt, Wg0, Wg1))
    for a, (Wh, bh, Wt, bt, W0, W1) in enumerate(combos):
        head = jnp.dot(enc, Wh[...], preferred_element_type=f32) + bh[...]
        tail = jnp.dot(enc, Wt[...], preferred_element_type=f32) + bt[...]
        heads_out[a] = head
        t_out[a, 0] = jnp.dot(tail, W0[...], preferred_element_type=f32)
        t_out[a, 1] = jnp.dot(tail, W1[...], preferred_element_type=f32)


def _tc_prep(aggp, h0, *ws):
    return pl.pallas_call(
        _tc_prep_body,
        out_shape=[
            jax.ShapeDtypeStruct((2, N, D_MODEL), jnp.float32),
            jax.ShapeDtypeStruct((2, 2, N, D_MODEL), jnp.float32),
        ],
    )(aggp, h0, *ws)


_RB = 256


def _tc_score_body(h_ref, t_ref, o_ref):
    h = h_ref[0]
    s0 = lax.dot_general(h, t_ref[0, 0],
                         dimension_numbers=(((1,), (1,)), ((), ())),
                         preferred_element_type=jnp.float32)
    s1 = lax.dot_general(h, t_ref[0, 1],
                         dimension_numbers=(((1,), (1,)), ((), ())),
                         preferred_element_type=jnp.float32)
    x = jnp.stack([s0.reshape(_RB, N // 128, 128),
                   s1.reshape(_RB, N // 128, 128)], axis=2)
    o_ref[0] = x.reshape(_RB, 2 * (N // 128), 128)


def _tc_scores(hcat, t):
    nrows = N + N_FIELDS
    nrt = (nrows + _RB - 1) // _RB
    out4 = pl.pallas_call(
        _tc_score_body,
        grid=(2, nrt),
        in_specs=[
            pl.BlockSpec((1, _RB, D_MODEL), lambda a, r: (a, r, 0)),
            pl.BlockSpec((1, 2, N, D_MODEL), lambda a, r: (a, 0, 0, 0)),
        ],
        out_specs=pl.BlockSpec((1, _RB, 2 * (N // 128), 128),
                               lambda a, r: (a, r, 0, 0)),
        out_shape=jax.ShapeDtypeStruct((2, nrows, 2 * (N // 128), 128),
                                       jnp.float32),
    )(hcat, t)
    res = out4.reshape(2, nrows, N // 128, 2, 128)
    res = res.transpose(0, 3, 1, 2, 4)
    return res.reshape(2, 2, nrows, N)


def kernel(input_ids, token_types, n_lower, n_upper, n_alpha, n_spaces,
           n_numeric, n_special, rx_ids, ry_ids, edge_index, edge_weights,
           we_table, nl_table, nu_table, na_table, nsp_table, nnum_table,
           nspec_table, tt_table, rx_table, ry_table, W_in, b_in, W_gcn,
           W_self, b_gcn, W_proj, b_proj, Ws_head, bs_head, Ws_tail, bs_tail,
           fields_s, Ws0, Ws1, Wg_head, bg_head, Wg_tail, bg_tail, fields_g,
           Wg0, Wg1):
    i32 = jnp.int32
    idxs = [x.astype(i32) for x in
            (input_ids, n_lower, n_upper, n_alpha, n_spaces, n_numeric,
             n_special, token_types, rx_ids, ry_ids)]
    tables = (we_table, nl_table, nu_table, na_table, nsp_table, nnum_table,
              nspec_table, tt_table, rx_table, ry_table)
    parts = _sc_embed(*idxs, *(t.T for t in tables))

    w3 = W_in.reshape(NFEAT, D_EPART, D_MODEL)
    h0 = _tc_h0(parts, w3, b_in.reshape(1, D_MODEL))

    src = edge_index[0].astype(i32)
    dst = edge_index[1].astype(i32)
    aggp = _sc_edges(h0, src, dst, edge_weights)

    heads, t = _tc_prep(
        aggp, h0, W_gcn, W_self, b_gcn.reshape(1, D_MODEL), W_proj,
        b_proj.reshape(1, D_MODEL),
        Ws_head, bs_head.reshape(1, D_MODEL), Ws_tail,
        bs_tail.reshape(1, D_MODEL), Ws0, Ws1,
        Wg_head, bg_head.reshape(1, D_MODEL), Wg_tail,
        bg_tail.reshape(1, D_MODEL), Wg0, Wg1)

    fields = jnp.stack([fields_s, fields_g], axis=0)
    hcat = jnp.concatenate([fields, heads], axis=1)
    return _tc_scores(hcat, t)


# speedup vs baseline: 4.0807x; 1.3672x over previous
import functools

import jax
import jax.numpy as jnp
from jax import lax
from jax.experimental import pallas as pl
from jax.experimental.pallas import tpu as pltpu
from jax.experimental.pallas import tpu_sc as plsc

N = 2048
E = 32768
VOCAB = 100000
U_TEXT = 512
U_DIST = 512
D_EPART = 12
D_GATHER = 16
D_MODEL = 128
N_FIELDS = 10
NFEAT = 10

NC = 2
NS = 16
L = 16
NW = NC * NS
NODES_PER_W = N // NW
EDGES_PER_W = E // NW
ECHUNK = 256
NECH = EDGES_PER_W // ECHUNK
ACC_ROWS_PER_TILE = N // NS

TABLE_ROWS = (VOCAB, U_TEXT, U_TEXT, U_TEXT, U_TEXT, U_TEXT, U_TEXT, 4,
              U_DIST, U_DIST)

@functools.cache
def _sc_mesh():
    return plsc.VectorSubcoreMesh(core_axis_name="c", subcore_axis_name="s",
                                  num_cores=NC, num_subcores=NS)


def _sc_embed_body(*refs):
    idx_refs = refs[0:NFEAT]
    tbl_refs = refs[NFEAT:2 * NFEAT]
    out = refs[2 * NFEAT]
    (ibuf, tbuf, ttbuf, wecols, pbuf,
     isem, tsem, dsem, osem) = refs[2 * NFEAT + 1:]
    c = lax.axis_index("c")
    s = lax.axis_index("s")
    base = (c * NS + s) * NODES_PER_W
    iota = lax.iota(jnp.int32, L)

    idescs = [pltpu.async_copy(idx_refs[f].at[pl.ds(base, NODES_PER_W)],
                               ibuf.at[f], isem) for f in range(NFEAT)]
    tdescs = [pltpu.async_copy(tbl_refs[k + 1],
                               ttbuf if k == 6 else tbuf.at[k if k < 6 else k - 1],
                               tsem)
              for k in range(NFEAT - 1)]
    for d in idescs:
        d.wait()

    wedescs = []
    for g in range(NODES_PER_W // L):
        v = ibuf[0, pl.ds(g * L, L)]
        v = jnp.minimum(jnp.maximum(v, 0), VOCAB - 1)
        for e in range(L):
            n = g * L + e
            wedescs.append(pltpu.async_copy(
                tbl_refs[0].at[:, pl.ds((v[e] >> 3) * 8, 8)],
                wecols.at[n], dsem))

    for d in tdescs:
        d.wait()

    for f in range(1, NFEAT):
        ub = TABLE_ROWS[f] - 1
        tt = f == 7

        def gath(g, _, f=f, ub=ub, tt=tt):
            idx16 = ibuf[f, pl.ds(g * L, L)]
            idx16 = jnp.minimum(jnp.maximum(idx16, 0), ub)
            kf = jnp.full((L,), f - 1 if f < 7 else f - 2, jnp.int32)
            for j in range(D_EPART):
                jf = jnp.full((L,), j, jnp.int32)
                if tt:
                    vals = plsc.load_gather(ttbuf, [jf, idx16])
                else:
                    vals = plsc.load_gather(tbuf, [kf, jf, idx16])
                pbuf[f, j, pl.ds(g * L, L)] = vals
            return 0
        lax.fori_loop(0, NODES_PER_W // L, gath, 0)

    for d in wedescs:
        d.wait()

    def we_extract(g, _):
        v = ibuf[0, pl.ds(g * L, L)]
        v = jnp.minimum(jnp.maximum(v, 0), VOCAB - 1)
        cv = v & 7
        nv = jnp.full((L,), g * L, jnp.int32) + iota
        for j in range(D_EPART):
            jf = jnp.full((L,), j, jnp.int32)
            pbuf[0, j, pl.ds(g * L, L)] = plsc.load_gather(
                wecols, [nv, jf, cv])
        return 0
    lax.fori_loop(0, NODES_PER_W // L, we_extract, 0)

    pltpu.async_copy(pbuf, out.at[:, :, pl.ds(base, NODES_PER_W)],
                     osem).wait()


@functools.cache
def _sc_embed_kernel():
    return pl.kernel(
        _sc_embed_body,
        out_type=jax.ShapeDtypeStruct((NFEAT, D_EPART, N), jnp.float32),
        mesh=_sc_mesh(),
        scratch_types=[
            pltpu.VMEM((NFEAT, NODES_PER_W), jnp.int32),
            pltpu.VMEM((NFEAT - 2, D_EPART, U_TEXT), jnp.float32),
            pltpu.VMEM((D_EPART, 4), jnp.float32),
            pltpu.VMEM((NODES_PER_W, D_EPART, 8), jnp.float32),
            pltpu.VMEM((NFEAT, D_EPART, NODES_PER_W), jnp.float32),
            pltpu.SemaphoreType.DMA,
            pltpu.SemaphoreType.DMA,
            pltpu.SemaphoreType.DMA,
            pltpu.SemaphoreType.DMA,
        ],
        compiler_params=pltpu.CompilerParams(needs_layout_passes=False,
                                             use_tc_tiling_on_sc=False),
    )


def _sc_embed(*args):
    return _sc_embed_kernel()(*args)


def _sc_edge_body(h0, srcr, dstr, ewr, out, acc, sidx, didx, wbuf, msg, obuf,
                  sem):
    c = lax.axis_index("c")
    s = lax.axis_index("s")

    def zrow(i, _):
        for j in range(D_MODEL // L):
            obuf[i, pl.ds(j * L, L)] = jnp.zeros((L,), jnp.float32)
        return 0
    lax.fori_loop(0, ACC_ROWS_PER_TILE, zrow, 0)
    pltpu.sync_copy(obuf, acc.at[pl.ds(s * ACC_ROWS_PER_TILE,
                                       ACC_ROWS_PER_TILE)])
    plsc.subcore_barrier()

    base = (c * NS + s) * EDGES_PER_W
    for ch in range(NECH):
        off = base + ch * ECHUNK
        pltpu.sync_copy(srcr.at[pl.ds(off, ECHUNK)], sidx)
        pltpu.sync_copy(dstr.at[pl.ds(off, ECHUNK)], didx)
        pltpu.sync_copy(ewr.at[pl.ds(off, ECHUNK)], wbuf)
        pltpu.async_copy(h0.at[sidx], msg, sem).wait()

        def scale_group(g, _):
            w16 = wbuf[pl.ds(g * L, L)]
            for e in range(L):
                wv = jnp.full((L,), w16[e], jnp.float32)
                i = g * L + e
                for j in range(D_MODEL // L):
                    msg[i, pl.ds(j * L, L)] = msg[i, pl.ds(j * L, L)] * wv
            return 0
        lax.fori_loop(0, ECHUNK // L, scale_group, 0)
        pltpu.sync_copy(msg, acc.at[didx], add=True)

    plsc.subcore_barrier()
    pltpu.sync_copy(acc.at[pl.ds(s * ACC_ROWS_PER_TILE, ACC_ROWS_PER_TILE)],
                    obuf)
    pltpu.sync_copy(obuf, out.at[c, pl.ds(s * ACC_ROWS_PER_TILE,
                                          ACC_ROWS_PER_TILE)])


@functools.cache
def _sc_edges_kernel():
    return pl.kernel(
        _sc_edge_body,
        out_type=jax.ShapeDtypeStruct((NC, N, D_MODEL), jnp.float32),
        mesh=_sc_mesh(),
        scratch_types=[
            pltpu.VMEM_SHARED((N, D_MODEL), jnp.float32),
            pltpu.VMEM((ECHUNK,), jnp.int32),
            pltpu.VMEM((ECHUNK,), jnp.int32),
            pltpu.VMEM((ECHUNK,), jnp.float32),
            pltpu.VMEM((ECHUNK, D_MODEL), jnp.float32),
            pltpu.VMEM((ACC_ROWS_PER_TILE, D_MODEL), jnp.float32),
            pltpu.SemaphoreType.DMA,
        ],
    )


def _sc_edges(*args):
    return _sc_edges_kernel()(*args)


def _tc_h0_body(parts_ref, w_ref, b_ref, out_ref):
    acc = None
    for f in range(NFEAT):
        t = lax.dot_general(parts_ref[f], w_ref[f],
                            dimension_numbers=(((0,), (0,)), ((), ())),
                            preferred_element_type=jnp.float32)
        acc = t if acc is None else acc + t
    out_ref[...] = acc + b_ref[...]


def _tc_h0(parts, w3, b2):
    return pl.pallas_call(
        _tc_h0_body,
        out_shape=jax.ShapeDtypeStruct((N, D_MODEL), jnp.float32),
    )(parts, w3, b2)


def _tc_prep_body(aggp, h0, Wg, Wslf, bg, Wp, bp,
                  Wsh, bsh, Wst, bst, Ws0, Ws1,
                  Wgh, bgh, Wgt, bgt, Wg0, Wg1,
                  heads_out, t_out):
    f32 = jnp.float32
    agg = aggp[0] + aggp[1]
    pre = (jnp.dot(agg, Wg[...], preferred_element_type=f32)
           + jnp.dot(h0[...], Wslf[...], preferred_element_type=f32)
           + bg[...])
    h = jnp.maximum(pre, 0.0)
    enc = jnp.dot(h, Wp[...], preferred_element_type=f32) + bp[...]
    combos = ((Wsh, bsh, Wst, bst, Ws0, Ws1),
              (Wgh, bgh, Wgt, bgt, Wg0, Wg1))
    for a, (Wh, bh, Wt, bt, W0, W1) in enumerate(combos):
        head = jnp.dot(enc, Wh[...], preferred_element_type=f32) + bh[...]
        tail = jnp.dot(enc, Wt[...], preferred_element_type=f32) + bt[...]
        heads_out[a] = head
        t_out[a, 0] = jnp.dot(tail, W0[...], preferred_element_type=f32)
        t_out[a, 1] = jnp.dot(tail, W1[...], preferred_element_type=f32)


def _tc_prep(aggp, h0, *ws):
    return pl.pallas_call(
        _tc_prep_body,
        out_shape=[
            jax.ShapeDtypeStruct((2, N, D_MODEL), jnp.float32),
            jax.ShapeDtypeStruct((2, 2, N, D_MODEL), jnp.float32),
        ],
    )(aggp, h0, *ws)


_RB = 256


def _tc_score_body(h_ref, t_ref, o_ref):
    h = h_ref[0]
    s0 = lax.dot_general(h, t_ref[0, 0],
                         dimension_numbers=(((1,), (1,)), ((), ())),
                         preferred_element_type=jnp.float32)
    s1 = lax.dot_general(h, t_ref[0, 1],
                         dimension_numbers=(((1,), (1,)), ((), ())),
                         preferred_element_type=jnp.float32)
    pieces = []
    for c1 in range(N // 128):
        pieces.append(s0[:, c1 * 128:(c1 + 1) * 128])
        pieces.append(s1[:, c1 * 128:(c1 + 1) * 128])
    p = jnp.stack(pieces, axis=0)
    o_ref[0] = p.transpose(1, 0, 2)


def _tc_scores(hcat, t):
    nrows = N + N_FIELDS
    nrt = (nrows + _RB - 1) // _RB
    out4 = pl.pallas_call(
        _tc_score_body,
        grid=(2, nrt),
        in_specs=[
            pl.BlockSpec((1, _RB, D_MODEL), lambda a, r: (a, r, 0)),
            pl.BlockSpec((1, 2, N, D_MODEL), lambda a, r: (a, 0, 0, 0)),
        ],
        out_specs=pl.BlockSpec((1, _RB, 2 * (N // 128), 128),
                               lambda a, r: (a, r, 0, 0)),
        out_shape=jax.ShapeDtypeStruct((2, nrows, 2 * (N // 128), 128),
                                       jnp.float32),
    )(hcat, t)
    res = out4.reshape(2, nrows, N // 128, 2, 128)
    res = res.transpose(0, 3, 1, 2, 4)
    return res.reshape(2, 2, nrows, N)


def kernel(input_ids, token_types, n_lower, n_upper, n_alpha, n_spaces,
           n_numeric, n_special, rx_ids, ry_ids, edge_index, edge_weights,
           we_table, nl_table, nu_table, na_table, nsp_table, nnum_table,
           nspec_table, tt_table, rx_table, ry_table, W_in, b_in, W_gcn,
           W_self, b_gcn, W_proj, b_proj, Ws_head, bs_head, Ws_tail, bs_tail,
           fields_s, Ws0, Ws1, Wg_head, bg_head, Wg_tail, bg_tail, fields_g,
           Wg0, Wg1):
    i32 = jnp.int32
    idxs = [x.astype(i32) for x in
            (input_ids, n_lower, n_upper, n_alpha, n_spaces, n_numeric,
             n_special, token_types, rx_ids, ry_ids)]
    tables = (we_table, nl_table, nu_table, na_table, nsp_table, nnum_table,
              nspec_table, tt_table, rx_table, ry_table)
    parts = _sc_embed(*idxs, *(t.T for t in tables))

    w3 = W_in.reshape(NFEAT, D_EPART, D_MODEL)
    h0 = _tc_h0(parts, w3, b_in.reshape(1, D_MODEL))

    src = edge_index[0].astype(i32)
    dst = edge_index[1].astype(i32)
    aggp = _sc_edges(h0, src, dst, edge_weights)

    heads, t = _tc_prep(
        aggp, h0, W_gcn, W_self, b_gcn.reshape(1, D_MODEL), W_proj,
        b_proj.reshape(1, D_MODEL),
        Ws_head, bs_head.reshape(1, D_MODEL), Ws_tail,
        bs_tail.reshape(1, D_MODEL), Ws0, Ws1,
        Wg_head, bg_head.reshape(1, D_MODEL), Wg_tail,
        bg_tail.reshape(1, D_MODEL), Wg0, Wg1)

    fields = jnp.stack([fields_s, fields_g], axis=0)
    hcat = jnp.concatenate([fields, heads], axis=1)
    return _tc_scores(hcat, t)


# speedup vs baseline: 4.4069x; 1.0800x over previous
import functools

import jax
import jax.numpy as jnp
from jax import lax
from jax.experimental import pallas as pl
from jax.experimental.pallas import tpu as pltpu
from jax.experimental.pallas import tpu_sc as plsc

N = 2048
E = 32768
VOCAB = 100000
U_TEXT = 512
U_DIST = 512
D_EPART = 12
D_GATHER = 16
D_MODEL = 128
N_FIELDS = 10
NFEAT = 10

NC = 2
NS = 16
L = 16
NW = NC * NS
NODES_PER_W = N // NW
EDGES_PER_W = E // NW
ECHUNK = 256
NECH = EDGES_PER_W // ECHUNK
ACC_ROWS_PER_TILE = N // NS

TABLE_ROWS = (VOCAB, U_TEXT, U_TEXT, U_TEXT, U_TEXT, U_TEXT, U_TEXT, 4,
              U_DIST, U_DIST)

@functools.cache
def _sc_mesh():
    return plsc.VectorSubcoreMesh(core_axis_name="c", subcore_axis_name="s",
                                  num_cores=NC, num_subcores=NS)


def _sc_embed_body(*refs):
    idx_refs = refs[0:NFEAT]
    tbl_refs = refs[NFEAT:2 * NFEAT]
    out = refs[2 * NFEAT]
    (ibuf, tbuf, ttbuf, wecols, pbuf,
     isem, tsem, dsem, osem) = refs[2 * NFEAT + 1:]
    c = lax.axis_index("c")
    s = lax.axis_index("s")
    base = (c * NS + s) * NODES_PER_W
    iota = lax.iota(jnp.int32, L)

    idescs = [pltpu.async_copy(idx_refs[f].at[pl.ds(base, NODES_PER_W)],
                               ibuf.at[f], isem) for f in range(NFEAT)]
    tdescs = [pltpu.async_copy(tbl_refs[k + 1],
                               ttbuf if k == 6 else tbuf.at[k if k < 6 else k - 1],
                               tsem)
              for k in range(NFEAT - 1)]
    for d in idescs:
        d.wait()

    wedescs = []
    for g in range(NODES_PER_W // L):
        v = ibuf[0, pl.ds(g * L, L)]
        v = jnp.minimum(jnp.maximum(v, 0), VOCAB - 1)
        for e in range(L):
            n = g * L + e
            wedescs.append(pltpu.async_copy(
                tbl_refs[0].at[:, pl.ds((v[e] >> 3) * 8, 8)],
                wecols.at[n], dsem))

    for d in tdescs:
        d.wait()

    for f in range(1, NFEAT):
        ub = TABLE_ROWS[f] - 1
        tt = f == 7

        def gath(g, _, f=f, ub=ub, tt=tt):
            idx16 = ibuf[f, pl.ds(g * L, L)]
            idx16 = jnp.minimum(jnp.maximum(idx16, 0), ub)
            kf = jnp.full((L,), f - 1 if f < 7 else f - 2, jnp.int32)
            for j in range(D_EPART):
                jf = jnp.full((L,), j, jnp.int32)
                if tt:
                    vals = plsc.load_gather(ttbuf, [jf, idx16])
                else:
                    vals = plsc.load_gather(tbuf, [kf, jf, idx16])
                pbuf[f, j, pl.ds(g * L, L)] = vals
            return 0
        lax.fori_loop(0, NODES_PER_W // L, gath, 0)

    for d in wedescs:
        d.wait()

    def we_extract(g, _):
        v = ibuf[0, pl.ds(g * L, L)]
        v = jnp.minimum(jnp.maximum(v, 0), VOCAB - 1)
        cv = v & 7
        nv = jnp.full((L,), g * L, jnp.int32) + iota
        for j in range(D_EPART):
            jf = jnp.full((L,), j, jnp.int32)
            pbuf[0, j, pl.ds(g * L, L)] = plsc.load_gather(
                wecols, [nv, jf, cv])
        return 0
    lax.fori_loop(0, NODES_PER_W // L, we_extract, 0)

    pltpu.async_copy(pbuf, out.at[:, :, pl.ds(base, NODES_PER_W)],
                     osem).wait()


@functools.cache
def _sc_embed_kernel():
    return pl.kernel(
        _sc_embed_body,
        out_type=jax.ShapeDtypeStruct((NFEAT, D_EPART, N), jnp.float32),
        mesh=_sc_mesh(),
        scratch_types=[
            pltpu.VMEM((NFEAT, NODES_PER_W), jnp.int32),
            pltpu.VMEM((NFEAT - 2, D_EPART, U_TEXT), jnp.float32),
            pltpu.VMEM((D_EPART, 4), jnp.float32),
            pltpu.VMEM((NODES_PER_W, D_EPART, 8), jnp.float32),
            pltpu.VMEM((NFEAT, D_EPART, NODES_PER_W), jnp.float32),
            pltpu.SemaphoreType.DMA,
            pltpu.SemaphoreType.DMA,
            pltpu.SemaphoreType.DMA,
            pltpu.SemaphoreType.DMA,
        ],
        compiler_params=pltpu.CompilerParams(needs_layout_passes=False,
                                             use_tc_tiling_on_sc=False),
    )


def _sc_embed(*args):
    return _sc_embed_kernel()(*args)


def _sc_edge_body(h0, srcr, dstr, ewr, out, acc, s0, s1, s2, s3,
                  d0, d1, d2, d3, wbuf, msg, obuf, isem, g0sem, g1sem,
                  s0sem, s1sem):
    sidx = (s0, s1, s2, s3)
    didx = (d0, d1, d2, d3)
    gsem = (g0sem, g1sem)
    ssem = (s0sem, s1sem)
    c = lax.axis_index("c")
    s = lax.axis_index("s")
    rows = pl.ds(s * ACC_ROWS_PER_TILE, ACC_ROWS_PER_TILE)

    base = (c * NS + s) * EDGES_PER_W
    idescs = []
    for ch in range(NECH):
        off = base + ch * ECHUNK
        idescs.append(pltpu.async_copy(srcr.at[pl.ds(off, ECHUNK)],
                                       sidx[ch], isem))
        idescs.append(pltpu.async_copy(dstr.at[pl.ds(off, ECHUNK)],
                                       didx[ch], isem))
        idescs.append(pltpu.async_copy(ewr.at[pl.ds(off, ECHUNK)],
                                       wbuf.at[ch], isem))

    def zrow(i, _):
        for j in range(D_MODEL // L):
            obuf[i, pl.ds(j * L, L)] = jnp.zeros((L,), jnp.float32)
        return 0
    lax.fori_loop(0, ACC_ROWS_PER_TILE, zrow, 0)
    pltpu.sync_copy(obuf, acc.at[rows])
    for d in idescs:
        d.wait()
    plsc.subcore_barrier()

    gd = [None, None]
    sd = [None, None]
    gd[0] = pltpu.async_copy(h0.at[sidx[0]], msg.at[0], gsem[0])
    for ch in range(NECH):
        nxt = ch + 1
        if nxt < NECH:
            if nxt >= 2:
                sd[nxt % 2].wait()
            gd[nxt % 2] = pltpu.async_copy(h0.at[sidx[nxt]],
                                           msg.at[nxt % 2], gsem[nxt % 2])
        gd[ch % 2].wait()

        def scale_group(g, _, ch=ch):
            w16 = wbuf[ch, pl.ds(g * L, L)]
            for e in range(L):
                wv = jnp.full((L,), w16[e], jnp.float32)
                i = g * L + e
                for j in range(D_MODEL // L):
                    msg[ch % 2, i, pl.ds(j * L, L)] = (
                        msg[ch % 2, i, pl.ds(j * L, L)] * wv)
            return 0
        lax.fori_loop(0, ECHUNK // L, scale_group, 0)
        sd[ch % 2] = pltpu.async_copy(msg.at[ch % 2], acc.at[didx[ch]],
                                      ssem[ch % 2], add=True)
    sd[(NECH - 2) % 2].wait()
    sd[(NECH - 1) % 2].wait()
    plsc.subcore_barrier()
    pltpu.sync_copy(acc.at[rows], obuf)
    pltpu.sync_copy(obuf, out.at[c, rows])


@functools.cache
def _sc_edges_kernel():
    return pl.kernel(
        _sc_edge_body,
        out_type=jax.ShapeDtypeStruct((NC, N, D_MODEL), jnp.float32),
        mesh=_sc_mesh(),
        scratch_types=[
            pltpu.VMEM_SHARED((N, D_MODEL), jnp.float32),
            *[pltpu.VMEM((ECHUNK,), jnp.int32) for _ in range(2 * NECH)],
            pltpu.VMEM((NECH, ECHUNK), jnp.float32),
            pltpu.VMEM((2, ECHUNK, D_MODEL), jnp.float32),
            pltpu.VMEM((ACC_ROWS_PER_TILE, D_MODEL), jnp.float32),
            pltpu.SemaphoreType.DMA,
            pltpu.SemaphoreType.DMA,
            pltpu.SemaphoreType.DMA,
            pltpu.SemaphoreType.DMA,
            pltpu.SemaphoreType.DMA,
        ],
    )


def _sc_edges(*args):
    return _sc_edges_kernel()(*args)


def _tc_h0_body(parts_ref, w_ref, b_ref, out_ref):
    acc = None
    for f in range(NFEAT):
        t = lax.dot_general(parts_ref[f], w_ref[f],
                            dimension_numbers=(((0,), (0,)), ((), ())),
                            preferred_element_type=jnp.float32)
        acc = t if acc is None else acc + t
    out_ref[...] = acc + b_ref[...]


def _tc_h0(parts, w3, b2):
    return pl.pallas_call(
        _tc_h0_body,
        out_shape=jax.ShapeDtypeStruct((N, D_MODEL), jnp.float32),
    )(parts, w3, b2)


def _tc_prep_body(aggp, h0, Wg, Wslf, bg, Wp, bp,
                  Wsh, bsh, Wst, bst, Ws0, Ws1,
                  Wgh, bgh, Wgt, bgt, Wg0, Wg1,
                  heads_out, t_out):
    f32 = jnp.float32
    agg = aggp[0] + aggp[1]
    pre = (jnp.dot(agg, Wg[...], preferred_element_type=f32)
           + jnp.dot(h0[...], Wslf[...], preferred_element_type=f32)
           + bg[...])
    h = jnp.maximum(pre, 0.0)
    enc = jnp.dot(h, Wp[...], preferred_element_type=f32) + bp[...]
    combos = ((Wsh, bsh, Wst, bst, Ws0, Ws1),
              (Wgh, bgh, Wgt, bgt, Wg0, Wg1))
    for a, (Wh, bh, Wt, bt, W0, W1) in enumerate(combos):
        head = jnp.dot(enc, Wh[...], preferred_element_type=f32) + bh[...]
        tail = jnp.dot(enc, Wt[...], preferred_element_type=f32) + bt[...]
        heads_out[a] = head
        t_out[a, 0] = jnp.dot(tail, W0[...], preferred_element_type=f32)
        t_out[a, 1] = jnp.dot(tail, W1[...], preferred_element_type=f32)


def _tc_prep(aggp, h0, *ws):
    return pl.pallas_call(
        _tc_prep_body,
        out_shape=[
            jax.ShapeDtypeStruct((2, N, D_MODEL), jnp.float32),
            jax.ShapeDtypeStruct((2, 2, N, D_MODEL), jnp.float32),
        ],
    )(aggp, h0, *ws)


_RB = 256


def _tc_score_body(h_ref, t_ref, o_ref):
    h = h_ref[0]
    s0 = lax.dot_general(h, t_ref[0, 0],
                         dimension_numbers=(((1,), (1,)), ((), ())),
                         preferred_element_type=jnp.float32)
    s1 = lax.dot_general(h, t_ref[0, 1],
                         dimension_numbers=(((1,), (1,)), ((), ())),
                         preferred_element_type=jnp.float32)
    pieces = []
    for c1 in range(N // 128):
        pieces.append(s0[:, c1 * 128:(c1 + 1) * 128])
        pieces.append(s1[:, c1 * 128:(c1 + 1) * 128])
    p = jnp.stack(pieces, axis=0)
    o_ref[0] = p.transpose(1, 0, 2)


def _tc_scores(hcat, t):
    nrows = N + N_FIELDS
    nrt = (nrows + _RB - 1) // _RB
    out4 = pl.pallas_call(
        _tc_score_body,
        grid=(2, nrt),
        in_specs=[
            pl.BlockSpec((1, _RB, D_MODEL), lambda a, r: (a, r, 0)),
            pl.BlockSpec((1, 2, N, D_MODEL), lambda a, r: (a, 0, 0, 0)),
        ],
        out_specs=pl.BlockSpec((1, _RB, 2 * (N // 128), 128),
                               lambda a, r: (a, r, 0, 0)),
        out_shape=jax.ShapeDtypeStruct((2, nrows, 2 * (N // 128), 128),
                                       jnp.float32),
    )(hcat, t)
    res = out4.reshape(2, nrows, N // 128, 2, 128)
    res = res.transpose(0, 3, 1, 2, 4)
    return res.reshape(2, 2, nrows, N)


def kernel(input_ids, token_types, n_lower, n_upper, n_alpha, n_spaces,
           n_numeric, n_special, rx_ids, ry_ids, edge_index, edge_weights,
           we_table, nl_table, nu_table, na_table, nsp_table, nnum_table,
           nspec_table, tt_table, rx_table, ry_table, W_in, b_in, W_gcn,
           W_self, b_gcn, W_proj, b_proj, Ws_head, bs_head, Ws_tail, bs_tail,
           fields_s, Ws0, Ws1, Wg_head, bg_head, Wg_tail, bg_tail, fields_g,
           Wg0, Wg1):
    i32 = jnp.int32
    idxs = [x.astype(i32) for x in
            (input_ids, n_lower, n_upper, n_alpha, n_spaces, n_numeric,
             n_special, token_types, rx_ids, ry_ids)]
    tables = (we_table, nl_table, nu_table, na_table, nsp_table, nnum_table,
              nspec_table, tt_table, rx_table, ry_table)
    parts = _sc_embed(*idxs, *(t.T for t in tables))

    w3 = W_in.reshape(NFEAT, D_EPART, D_MODEL)
    h0 = _tc_h0(parts, w3, b_in.reshape(1, D_MODEL))

    src = edge_index[0].astype(i32)
    dst = edge_index[1].astype(i32)
    aggp = _sc_edges(h0, src, dst, edge_weights)

    heads, t = _tc_prep(
        aggp, h0, W_gcn, W_self, b_gcn.reshape(1, D_MODEL), W_proj,
        b_proj.reshape(1, D_MODEL),
        Ws_head, bs_head.reshape(1, D_MODEL), Ws_tail,
        bs_tail.reshape(1, D_MODEL), Ws0, Ws1,
        Wg_head, bg_head.reshape(1, D_MODEL), Wg_tail,
        bg_tail.reshape(1, D_MODEL), Wg0, Wg1)

    fields = jnp.stack([fields_s, fields_g], axis=0)
    hcat = jnp.concatenate([fields, heads], axis=1)
    return _tc_scores(hcat, t)


# speedup vs baseline: 4.6575x; 1.0568x over previous
import functools

import jax
import jax.numpy as jnp
from jax import lax
from jax.experimental import pallas as pl
from jax.experimental.pallas import tpu as pltpu
from jax.experimental.pallas import tpu_sc as plsc

N = 2048
E = 32768
VOCAB = 100000
U_TEXT = 512
U_DIST = 512
D_EPART = 12
D_GATHER = 16
D_MODEL = 128
N_FIELDS = 10
NFEAT = 10

NC = 2
NS = 16
L = 16
NW = NC * NS
NODES_PER_W = N // NW
EDGES_PER_W = E // NW
ECHUNK = 256
NECH = EDGES_PER_W // ECHUNK
ACC_ROWS_PER_TILE = N // NS

TABLE_ROWS = (VOCAB, U_TEXT, U_TEXT, U_TEXT, U_TEXT, U_TEXT, U_TEXT, 4,
              U_DIST, U_DIST)

@functools.cache
def _sc_mesh():
    return plsc.VectorSubcoreMesh(core_axis_name="c", subcore_axis_name="s",
                                  num_cores=NC, num_subcores=NS)


def _sc_embed_body(*refs):
    idx_refs = refs[0:NFEAT]
    tbl_refs = refs[NFEAT:2 * NFEAT]
    out = refs[2 * NFEAT]
    (ibuf, tbuf, ttbuf, wecols, pbuf,
     isem, tsem, dsem, osem) = refs[2 * NFEAT + 1:]
    c = lax.axis_index("c")
    s = lax.axis_index("s")
    base = (c * NS + s) * NODES_PER_W
    iota = lax.iota(jnp.int32, L)

    idescs = [pltpu.async_copy(idx_refs[f].at[pl.ds(base, NODES_PER_W)],
                               ibuf.at[f], isem) for f in range(NFEAT)]
    tdescs = [pltpu.async_copy(tbl_refs[k + 1],
                               ttbuf if k == 6 else tbuf.at[k if k < 6 else k - 1],
                               tsem)
              for k in range(NFEAT - 1)]
    for d in idescs:
        d.wait()

    wedescs = []
    for g in range(NODES_PER_W // L):
        v = ibuf[0, pl.ds(g * L, L)]
        v = jnp.minimum(jnp.maximum(v, 0), VOCAB - 1)
        for e in range(L):
            n = g * L + e
            wedescs.append(pltpu.async_copy(
                tbl_refs[0].at[:, pl.ds((v[e] >> 3) * 8, 8)],
                wecols.at[n], dsem))

    for d in tdescs:
        d.wait()

    for f in range(1, NFEAT):
        ub = TABLE_ROWS[f] - 1
        tt = f == 7

        def gath(g, _, f=f, ub=ub, tt=tt):
            idx16 = ibuf[f, pl.ds(g * L, L)]
            idx16 = jnp.minimum(jnp.maximum(idx16, 0), ub)
            kf = jnp.full((L,), f - 1 if f < 7 else f - 2, jnp.int32)
            for j in range(D_EPART):
                jf = jnp.full((L,), j, jnp.int32)
                if tt:
                    vals = plsc.load_gather(ttbuf, [jf, idx16])
                else:
                    vals = plsc.load_gather(tbuf, [kf, jf, idx16])
                pbuf[f, j, pl.ds(g * L, L)] = vals
            return 0
        lax.fori_loop(0, NODES_PER_W // L, gath, 0)

    for d in wedescs:
        d.wait()

    def we_extract(g, _):
        v = ibuf[0, pl.ds(g * L, L)]
        v = jnp.minimum(jnp.maximum(v, 0), VOCAB - 1)
        cv = v & 7
        nv = jnp.full((L,), g * L, jnp.int32) + iota
        for j in range(D_EPART):
            jf = jnp.full((L,), j, jnp.int32)
            pbuf[0, j, pl.ds(g * L, L)] = plsc.load_gather(
                wecols, [nv, jf, cv])
        return 0
    lax.fori_loop(0, NODES_PER_W // L, we_extract, 0)

    pltpu.async_copy(pbuf, out.at[:, :, pl.ds(base, NODES_PER_W)],
                     osem).wait()


@functools.cache
def _sc_embed_kernel():
    return pl.kernel(
        _sc_embed_body,
        out_type=jax.ShapeDtypeStruct((NFEAT, D_EPART, N), jnp.float32),
        mesh=_sc_mesh(),
        scratch_types=[
            pltpu.VMEM((NFEAT, NODES_PER_W), jnp.int32),
            pltpu.VMEM((NFEAT - 2, D_EPART, U_TEXT), jnp.float32),
            pltpu.VMEM((D_EPART, 4), jnp.float32),
            pltpu.VMEM((NODES_PER_W, D_EPART, 8), jnp.float32),
            pltpu.VMEM((NFEAT, D_EPART, NODES_PER_W), jnp.float32),
            pltpu.SemaphoreType.DMA,
            pltpu.SemaphoreType.DMA,
            pltpu.SemaphoreType.DMA,
            pltpu.SemaphoreType.DMA,
        ],
        compiler_params=pltpu.CompilerParams(needs_layout_passes=False,
                                             use_tc_tiling_on_sc=False),
    )


def _sc_embed(*args):
    return _sc_embed_kernel()(*args)


def _sc_edge_body(h0, srcr, dstr, ewr, out, acc, s0, s1, s2, s3,
                  d0, d1, d2, d3, wbuf, msg, obuf, isem, g0sem, g1sem,
                  s0sem, s1sem):
    sidx = (s0, s1, s2, s3)
    didx = (d0, d1, d2, d3)
    gsem = (g0sem, g1sem)
    ssem = (s0sem, s1sem)
    c = lax.axis_index("c")
    s = lax.axis_index("s")
    rows = pl.ds(s * ACC_ROWS_PER_TILE, ACC_ROWS_PER_TILE)

    base = (c * NS + s) * EDGES_PER_W
    idescs = []
    for ch in range(NECH):
        off = base + ch * ECHUNK
        idescs.append(pltpu.async_copy(srcr.at[pl.ds(off, ECHUNK)],
                                       sidx[ch], isem))
        idescs.append(pltpu.async_copy(dstr.at[pl.ds(off, ECHUNK)],
                                       didx[ch], isem))
        idescs.append(pltpu.async_copy(ewr.at[pl.ds(off, ECHUNK)],
                                       wbuf.at[ch], isem))

    def zrow(i, _):
        for j in range(D_MODEL // L):
            obuf[i, pl.ds(j * L, L)] = jnp.zeros((L,), jnp.float32)
        return 0
    lax.fori_loop(0, ACC_ROWS_PER_TILE, zrow, 0)
    pltpu.sync_copy(obuf, acc.at[rows])
    for d in idescs:
        d.wait()
    plsc.subcore_barrier()

    gd = [None, None]
    sd = [None, None]
    gd[0] = pltpu.async_copy(h0.at[sidx[0]], msg.at[0], gsem[0])
    for ch in range(NECH):
        nxt = ch + 1
        if nxt < NECH:
            if nxt >= 2:
                sd[nxt % 2].wait()
            gd[nxt % 2] = pltpu.async_copy(h0.at[sidx[nxt]],
                                           msg.at[nxt % 2], gsem[nxt % 2])
        gd[ch % 2].wait()

        def scale_group(g, _, ch=ch):
            w16 = wbuf[ch, pl.ds(g * L, L)]
            for e in range(L):
                wv = jnp.full((L,), w16[e], jnp.float32)
                i = g * L + e
                for j in range(D_MODEL // L):
                    msg[ch % 2, i, pl.ds(j * L, L)] = (
                        msg[ch % 2, i, pl.ds(j * L, L)] * wv)
            return 0
        lax.fori_loop(0, ECHUNK // L, scale_group, 0)
        sd[ch % 2] = pltpu.async_copy(msg.at[ch % 2], acc.at[didx[ch]],
                                      ssem[ch % 2], add=True)
    sd[(NECH - 2) % 2].wait()
    sd[(NECH - 1) % 2].wait()
    plsc.subcore_barrier()
    pltpu.sync_copy(acc.at[rows], obuf)
    pltpu.sync_copy(obuf, out.at[c, rows])


@functools.cache
def _sc_edges_kernel():
    return pl.kernel(
        _sc_edge_body,
        out_type=jax.ShapeDtypeStruct((NC, N, D_MODEL), jnp.float32),
        mesh=_sc_mesh(),
        scratch_types=[
            pltpu.VMEM_SHARED((N, D_MODEL), jnp.float32),
            *[pltpu.VMEM((ECHUNK,), jnp.int32) for _ in range(2 * NECH)],
            pltpu.VMEM((NECH, ECHUNK), jnp.float32),
            pltpu.VMEM((2, ECHUNK, D_MODEL), jnp.float32),
            pltpu.VMEM((ACC_ROWS_PER_TILE, D_MODEL), jnp.float32),
            pltpu.SemaphoreType.DMA,
            pltpu.SemaphoreType.DMA,
            pltpu.SemaphoreType.DMA,
            pltpu.SemaphoreType.DMA,
            pltpu.SemaphoreType.DMA,
        ],
    )


def _sc_edges(*args):
    return _sc_edges_kernel()(*args)


def _tc_h0_body(parts_ref, w_ref, b_ref, out_ref):
    acc = None
    for f in range(NFEAT):
        t = lax.dot_general(parts_ref[f], w_ref[f],
                            dimension_numbers=(((0,), (0,)), ((), ())),
                            preferred_element_type=jnp.float32)
        acc = t if acc is None else acc + t
    out_ref[...] = acc + b_ref[...]


def _tc_h0(parts, w3, b2):
    return pl.pallas_call(
        _tc_h0_body,
        out_shape=jax.ShapeDtypeStruct((N, D_MODEL), jnp.float32),
    )(parts, w3, b2)


def _tc_prep_body(aggp, h0, fields, Wg, Wslf, bg, Wp, bp,
                  Wsh, bsh, Wst, bst, Ws0, Ws1,
                  Wgh, bgh, Wgt, bgt, Wg0, Wg1,
                  hcat_out, t_out):
    f32 = jnp.float32
    agg = aggp[0] + aggp[1]
    pre = (jnp.dot(agg, Wg[...], preferred_element_type=f32)
           + jnp.dot(h0[...], Wslf[...], preferred_element_type=f32)
           + bg[...])
    h = jnp.maximum(pre, 0.0)
    enc = jnp.dot(h, Wp[...], preferred_element_type=f32) + bp[...]
    combos = ((Wsh, bsh, Wst, bst, Ws0, Ws1),
              (Wgh, bgh, Wgt, bgt, Wg0, Wg1))
    for a, (Wh, bh, Wt, bt, W0, W1) in enumerate(combos):
        head = jnp.dot(enc, Wh[...], preferred_element_type=f32) + bh[...]
        tail = jnp.dot(enc, Wt[...], preferred_element_type=f32) + bt[...]
        hcat_out[a] = jnp.concatenate([fields[a], head], axis=0)
        t_out[a, 0] = jnp.dot(tail, W0[...], preferred_element_type=f32)
        t_out[a, 1] = jnp.dot(tail, W1[...], preferred_element_type=f32)


def _tc_prep(aggp, h0, fields, *ws):
    return pl.pallas_call(
        _tc_prep_body,
        out_shape=[
            jax.ShapeDtypeStruct((2, N + N_FIELDS, D_MODEL), jnp.float32),
            jax.ShapeDtypeStruct((2, 2, N, D_MODEL), jnp.float32),
        ],
    )(aggp, h0, fields, *ws)


_RB = 344


def _tc_score_body(h_ref, t_ref, o_ref):
    h = h_ref[0]
    s0 = lax.dot_general(h, t_ref[0, 0],
                         dimension_numbers=(((1,), (1,)), ((), ())),
                         preferred_element_type=jnp.float32)
    s1 = lax.dot_general(h, t_ref[0, 1],
                         dimension_numbers=(((1,), (1,)), ((), ())),
                         preferred_element_type=jnp.float32)
    pieces = []
    for c1 in range(N // 128):
        pieces.append(s0[:, c1 * 128:(c1 + 1) * 128])
        pieces.append(s1[:, c1 * 128:(c1 + 1) * 128])
    p = jnp.stack(pieces, axis=0)
    o_ref[0] = p.transpose(1, 0, 2)


def _tc_scores(hcat, t):
    nrows = N + N_FIELDS
    nrt = (nrows + _RB - 1) // _RB
    out4 = pl.pallas_call(
        _tc_score_body,
        grid=(2, nrt),
        in_specs=[
            pl.BlockSpec((1, _RB, D_MODEL), lambda a, r: (a, r, 0)),
            pl.BlockSpec((1, 2, N, D_MODEL), lambda a, r: (a, 0, 0, 0)),
        ],
        out_specs=pl.BlockSpec((1, _RB, 2 * (N // 128), 128),
                               lambda a, r: (a, r, 0, 0)),
        out_shape=jax.ShapeDtypeStruct((2, nrows, 2 * (N // 128), 128),
                                       jnp.float32),
    )(hcat, t)
    res = out4.reshape(2, nrows, N // 128, 2, 128)
    res = res.transpose(0, 3, 1, 2, 4)
    return res.reshape(2, 2, nrows, N)


def kernel(input_ids, token_types, n_lower, n_upper, n_alpha, n_spaces,
           n_numeric, n_special, rx_ids, ry_ids, edge_index, edge_weights,
           we_table, nl_table, nu_table, na_table, nsp_table, nnum_table,
           nspec_table, tt_table, rx_table, ry_table, W_in, b_in, W_gcn,
           W_self, b_gcn, W_proj, b_proj, Ws_head, bs_head, Ws_tail, bs_tail,
           fields_s, Ws0, Ws1, Wg_head, bg_head, Wg_tail, bg_tail, fields_g,
           Wg0, Wg1):
    i32 = jnp.int32
    idxs = [x.astype(i32) for x in
            (input_ids, n_lower, n_upper, n_alpha, n_spaces, n_numeric,
             n_special, token_types, rx_ids, ry_ids)]
    tables = (we_table, nl_table, nu_table, na_table, nsp_table, nnum_table,
              nspec_table, tt_table, rx_table, ry_table)
    parts = _sc_embed(*idxs, *(t.T for t in tables))

    w3 = W_in.reshape(NFEAT, D_EPART, D_MODEL)
    h0 = _tc_h0(parts, w3, b_in.reshape(1, D_MODEL))

    src = edge_index[0].astype(i32)
    dst = edge_index[1].astype(i32)
    aggp = _sc_edges(h0, src, dst, edge_weights)

    fields = jnp.stack([fields_s, fields_g], axis=0)
    hcat, t = _tc_prep(
        aggp, h0, fields, W_gcn, W_self, b_gcn.reshape(1, D_MODEL), W_proj,
        b_proj.reshape(1, D_MODEL),
        Ws_head, bs_head.reshape(1, D_MODEL), Ws_tail,
        bs_tail.reshape(1, D_MODEL), Ws0, Ws1,
        Wg_head, bg_head.reshape(1, D_MODEL), Wg_tail,
        bg_tail.reshape(1, D_MODEL), Wg0, Wg1)

    return _tc_scores(hcat, t)


# speedup vs baseline: 4.9110x; 1.0544x over previous
import functools

import jax
import jax.numpy as jnp
from jax import lax
from jax.experimental import pallas as pl
from jax.experimental.pallas import tpu as pltpu
from jax.experimental.pallas import tpu_sc as plsc

N = 2048
E = 32768
VOCAB = 100000
U_TEXT = 512
U_DIST = 512
D_EPART = 12
D_GATHER = 16
D_MODEL = 128
N_FIELDS = 10
NFEAT = 10

NC = 2
NS = 16
L = 16
NW = NC * NS
NODES_PER_W = N // NW
EDGES_PER_W = E // NW
ECHUNK = 256
NECH = EDGES_PER_W // ECHUNK
ACC_ROWS_PER_TILE = N // NS

TABLE_ROWS = (VOCAB, U_TEXT, U_TEXT, U_TEXT, U_TEXT, U_TEXT, U_TEXT, 4,
              U_DIST, U_DIST)
_SMALL_OFF = [None]
_acc = 0
for _r in TABLE_ROWS[1:]:
    _SMALL_OFF.append(_acc)
    _acc += D_EPART * _r
_SMALL_TOTAL = _acc

@functools.cache
def _sc_mesh():
    return plsc.VectorSubcoreMesh(core_axis_name="c", subcore_axis_name="s",
                                  num_cores=NC, num_subcores=NS)


def _sc_embed_body(*refs):
    idx_refs = refs[0:NFEAT]
    wet = refs[NFEAT]
    smalls = refs[NFEAT + 1]
    out = refs[NFEAT + 2]
    (ibuf, tflat, wecols, pbuf,
     isem, tsem, dsem, osem) = refs[NFEAT + 3:]
    c = lax.axis_index("c")
    s = lax.axis_index("s")
    base = (c * NS + s) * NODES_PER_W
    iota = lax.iota(jnp.int32, L)

    idescs = [pltpu.async_copy(idx_refs[f].at[pl.ds(base, NODES_PER_W)],
                               ibuf.at[f], isem) for f in range(NFEAT)]
    tdesc = pltpu.async_copy(smalls, tflat, tsem)
    for d in idescs:
        d.wait()

    wedescs = []
    for g in range(NODES_PER_W // L):
        v = ibuf[0, pl.ds(g * L, L)]
        v = jnp.minimum(jnp.maximum(v, 0), VOCAB - 1)
        for e in range(L):
            n = g * L + e
            wedescs.append(pltpu.async_copy(
                wet.at[:, pl.ds((v[e] >> 3) * 8, 8)],
                wecols.at[n], dsem))

    tdesc.wait()

    for f in range(1, NFEAT):
        ub = TABLE_ROWS[f] - 1
        off = _SMALL_OFF[f]
        stride = ub + 1

        def gath(g, _, ub=ub, off=off, stride=stride, f=f):
            idx16 = ibuf[f, pl.ds(g * L, L)]
            idx16 = jnp.minimum(jnp.maximum(idx16, 0), ub)
            for j in range(D_EPART):
                flat = idx16 + (off + j * stride)
                pbuf[f, j, pl.ds(g * L, L)] = plsc.load_gather(tflat, [flat])
            return 0
        lax.fori_loop(0, NODES_PER_W // L, gath, 0)

    for d in wedescs:
        d.wait()

    def we_extract(g, _):
        v = ibuf[0, pl.ds(g * L, L)]
        v = jnp.minimum(jnp.maximum(v, 0), VOCAB - 1)
        cv = v & 7
        nv = jnp.full((L,), g * L, jnp.int32) + iota
        for j in range(D_EPART):
            jf = jnp.full((L,), j, jnp.int32)
            pbuf[0, j, pl.ds(g * L, L)] = plsc.load_gather(
                wecols, [nv, jf, cv])
        return 0
    lax.fori_loop(0, NODES_PER_W // L, we_extract, 0)

    pltpu.async_copy(pbuf, out.at[:, :, pl.ds(base, NODES_PER_W)],
                     osem).wait()


@functools.cache
def _sc_embed_kernel():
    return pl.kernel(
        _sc_embed_body,
        out_type=jax.ShapeDtypeStruct((NFEAT, D_EPART, N), jnp.float32),
        mesh=_sc_mesh(),
        scratch_types=[
            pltpu.VMEM((NFEAT, NODES_PER_W), jnp.int32),
            pltpu.VMEM((_SMALL_TOTAL,), jnp.float32),
            pltpu.VMEM((NODES_PER_W, D_EPART, 8), jnp.float32),
            pltpu.VMEM((NFEAT, D_EPART, NODES_PER_W), jnp.float32),
            pltpu.SemaphoreType.DMA,
            pltpu.SemaphoreType.DMA,
            pltpu.SemaphoreType.DMA,
            pltpu.SemaphoreType.DMA,
        ],
        compiler_params=pltpu.CompilerParams(needs_layout_passes=False,
                                             use_tc_tiling_on_sc=False),
    )


def _sc_embed(*args):
    return _sc_embed_kernel()(*args)


def _sc_edge_body(h0, srcr, dstr, ewr, out, acc, s0, s1, s2, s3,
                  d0, d1, d2, d3, wbuf, msg, obuf, isem, g0sem, g1sem,
                  s0sem, s1sem):
    sidx = (s0, s1, s2, s3)
    didx = (d0, d1, d2, d3)
    gsem = (g0sem, g1sem)
    ssem = (s0sem, s1sem)
    c = lax.axis_index("c")
    s = lax.axis_index("s")
    rows = pl.ds(s * ACC_ROWS_PER_TILE, ACC_ROWS_PER_TILE)

    base = (c * NS + s) * EDGES_PER_W
    idescs = []
    for ch in range(NECH):
        off = base + ch * ECHUNK
        idescs.append(pltpu.async_copy(srcr.at[pl.ds(off, ECHUNK)],
                                       sidx[ch], isem))
        idescs.append(pltpu.async_copy(dstr.at[pl.ds(off, ECHUNK)],
                                       didx[ch], isem))
        idescs.append(pltpu.async_copy(ewr.at[pl.ds(off, ECHUNK)],
                                       wbuf.at[ch], isem))

    def zrow(i, _):
        for j in range(D_MODEL // L):
            obuf[i, pl.ds(j * L, L)] = jnp.zeros((L,), jnp.float32)
        return 0
    lax.fori_loop(0, ACC_ROWS_PER_TILE, zrow, 0)
    pltpu.sync_copy(obuf, acc.at[rows])
    for d in idescs:
        d.wait()
    plsc.subcore_barrier()

    gd = [None, None]
    sd = [None, None]
    gd[0] = pltpu.async_copy(h0.at[sidx[0]], msg.at[0], gsem[0])
    for ch in range(NECH):
        nxt = ch + 1
        if nxt < NECH:
            if nxt >= 2:
                sd[nxt % 2].wait()
            gd[nxt % 2] = pltpu.async_copy(h0.at[sidx[nxt]],
                                           msg.at[nxt % 2], gsem[nxt % 2])
        gd[ch % 2].wait()

        def scale_group(g, _, ch=ch):
            w16 = wbuf[ch, pl.ds(g * L, L)]
            for e in range(L):
                wv = jnp.full((L,), w16[e], jnp.float32)
                i = g * L + e
                for j in range(D_MODEL // L):
                    msg[ch % 2, i, pl.ds(j * L, L)] = (
                        msg[ch % 2, i, pl.ds(j * L, L)] * wv)
            return 0
        lax.fori_loop(0, ECHUNK // L, scale_group, 0)
        sd[ch % 2] = pltpu.async_copy(msg.at[ch % 2], acc.at[didx[ch]],
                                      ssem[ch % 2], add=True)
    sd[(NECH - 2) % 2].wait()
    sd[(NECH - 1) % 2].wait()
    plsc.subcore_barrier()
    pltpu.sync_copy(acc.at[rows], obuf)
    pltpu.sync_copy(obuf, out.at[c, rows])


@functools.cache
def _sc_edges_kernel():
    return pl.kernel(
        _sc_edge_body,
        out_type=jax.ShapeDtypeStruct((NC, N, D_MODEL), jnp.float32),
        mesh=_sc_mesh(),
        scratch_types=[
            pltpu.VMEM_SHARED((N, D_MODEL), jnp.float32),
            *[pltpu.VMEM((ECHUNK,), jnp.int32) for _ in range(2 * NECH)],
            pltpu.VMEM((NECH, ECHUNK), jnp.float32),
            pltpu.VMEM((2, ECHUNK, D_MODEL), jnp.float32),
            pltpu.VMEM((ACC_ROWS_PER_TILE, D_MODEL), jnp.float32),
            pltpu.SemaphoreType.DMA,
            pltpu.SemaphoreType.DMA,
            pltpu.SemaphoreType.DMA,
            pltpu.SemaphoreType.DMA,
            pltpu.SemaphoreType.DMA,
        ],
    )


def _sc_edges(*args):
    return _sc_edges_kernel()(*args)


def _tc_h0_body(parts_ref, w_ref, b_ref, out_ref):
    acc = None
    for f in range(NFEAT):
        t = lax.dot_general(parts_ref[f], w_ref[f],
                            dimension_numbers=(((0,), (0,)), ((), ())),
                            preferred_element_type=jnp.float32)
        acc = t if acc is None else acc + t
    out_ref[...] = acc + b_ref[...]


def _tc_h0(parts, w3, b2):
    return pl.pallas_call(
        _tc_h0_body,
        out_shape=jax.ShapeDtypeStruct((N, D_MODEL), jnp.float32),
    )(parts, w3, b2)


def _tc_prep_body(aggp, h0, fields, Wg, Wslf, bg, Wp, bp,
                  Wsh, bsh, Wst, bst, Ws0, Ws1,
                  Wgh, bgh, Wgt, bgt, Wg0, Wg1,
                  hcat_out, t_out):
    f32 = jnp.float32
    agg = aggp[0] + aggp[1]
    pre = (jnp.dot(agg, Wg[...], preferred_element_type=f32)
           + jnp.dot(h0[...], Wslf[...], preferred_element_type=f32)
           + bg[...])
    h = jnp.maximum(pre, 0.0)
    enc = jnp.dot(h, Wp[...], preferred_element_type=f32) + bp[...]
    combos = ((Wsh, bsh, Wst, bst, Ws0, Ws1),
              (Wgh, bgh, Wgt, bgt, Wg0, Wg1))
    for a, (Wh, bh, Wt, bt, W0, W1) in enumerate(combos):
        head = jnp.dot(enc, Wh[...], preferred_element_type=f32) + bh[...]
        tail = jnp.dot(enc, Wt[...], preferred_element_type=f32) + bt[...]
        hcat_out[a] = jnp.concatenate([fields[a], head], axis=0)
        t_out[a, 0] = jnp.dot(tail, W0[...], preferred_element_type=f32)
        t_out[a, 1] = jnp.dot(tail, W1[...], preferred_element_type=f32)


def _tc_prep(aggp, h0, fields, *ws):
    return pl.pallas_call(
        _tc_prep_body,
        out_shape=[
            jax.ShapeDtypeStruct((2, N + N_FIELDS, D_MODEL), jnp.float32),
            jax.ShapeDtypeStruct((2, 2, N, D_MODEL), jnp.float32),
        ],
    )(aggp, h0, fields, *ws)


_RB = 344


def _tc_score_body(h_ref, t_ref, o_ref):
    h = h_ref[0]
    s0 = lax.dot_general(h, t_ref[0, 0],
                         dimension_numbers=(((1,), (1,)), ((), ())),
                         preferred_element_type=jnp.float32)
    s1 = lax.dot_general(h, t_ref[0, 1],
                         dimension_numbers=(((1,), (1,)), ((), ())),
                         preferred_element_type=jnp.float32)
    pieces = []
    for c1 in range(N // 128):
        pieces.append(s0[:, c1 * 128:(c1 + 1) * 128])
        pieces.append(s1[:, c1 * 128:(c1 + 1) * 128])
    p = jnp.stack(pieces, axis=0)
    o_ref[0] = p.transpose(1, 0, 2)


def _tc_scores(hcat, t):
    nrows = N + N_FIELDS
    nrt = (nrows + _RB - 1) // _RB
    out4 = pl.pallas_call(
        _tc_score_body,
        grid=(2, nrt),
        in_specs=[
            pl.BlockSpec((1, _RB, D_MODEL), lambda a, r: (a, r, 0)),
            pl.BlockSpec((1, 2, N, D_MODEL), lambda a, r: (a, 0, 0, 0)),
        ],
        out_specs=pl.BlockSpec((1, _RB, 2 * (N // 128), 128),
                               lambda a, r: (a, r, 0, 0)),
        out_shape=jax.ShapeDtypeStruct((2, nrows, 2 * (N // 128), 128),
                                       jnp.float32),
    )(hcat, t)
    res = out4.reshape(2, nrows, N // 128, 2, 128)
    res = res.transpose(0, 3, 1, 2, 4)
    return res.reshape(2, 2, nrows, N)


def kernel(input_ids, token_types, n_lower, n_upper, n_alpha, n_spaces,
           n_numeric, n_special, rx_ids, ry_ids, edge_index, edge_weights,
           we_table, nl_table, nu_table, na_table, nsp_table, nnum_table,
           nspec_table, tt_table, rx_table, ry_table, W_in, b_in, W_gcn,
           W_self, b_gcn, W_proj, b_proj, Ws_head, bs_head, Ws_tail, bs_tail,
           fields_s, Ws0, Ws1, Wg_head, bg_head, Wg_tail, bg_tail, fields_g,
           Wg0, Wg1):
    i32 = jnp.int32
    idxs = [x.astype(i32) for x in
            (input_ids, n_lower, n_upper, n_alpha, n_spaces, n_numeric,
             n_special, token_types, rx_ids, ry_ids)]
    small_tables = (nl_table, nu_table, na_table, nsp_table, nnum_table,
                    nspec_table, tt_table, rx_table, ry_table)
    smalls = jnp.concatenate([t.T.reshape(-1) for t in small_tables])
    parts = _sc_embed(*idxs, we_table.T, smalls)

    w3 = W_in.reshape(NFEAT, D_EPART, D_MODEL)
    h0 = _tc_h0(parts, w3, b_in.reshape(1, D_MODEL))

    src = edge_index[0].astype(i32)
    dst = edge_index[1].astype(i32)
    aggp = _sc_edges(h0, src, dst, edge_weights)

    fields = jnp.stack([fields_s, fields_g], axis=0)
    hcat, t = _tc_prep(
        aggp, h0, fields, W_gcn, W_self, b_gcn.reshape(1, D_MODEL), W_proj,
        b_proj.reshape(1, D_MODEL),
        Ws_head, bs_head.reshape(1, D_MODEL), Ws_tail,
        bs_tail.reshape(1, D_MODEL), Ws0, Ws1,
        Wg_head, bg_head.reshape(1, D_MODEL), Wg_tail,
        bg_tail.reshape(1, D_MODEL), Wg0, Wg1)

    return _tc_scores(hcat, t)


# speedup vs baseline: 4.9437x; 1.0067x over previous
import functools

import jax
import jax.numpy as jnp
from jax import lax
from jax.experimental import pallas as pl
from jax.experimental.pallas import tpu as pltpu
from jax.experimental.pallas import tpu_sc as plsc

N = 2048
E = 32768
VOCAB = 100000
U_TEXT = 512
U_DIST = 512
D_EPART = 12
D_GATHER = 16
D_MODEL = 128
N_FIELDS = 10
NFEAT = 10

NC = 2
NS = 16
L = 16
NW = NC * NS
NODES_PER_W = N // NW
EDGES_PER_W = E // NW
ECHUNK = 256
NECH = EDGES_PER_W // ECHUNK
ACC_ROWS_PER_TILE = N // NS

TABLE_ROWS = (VOCAB, U_TEXT, U_TEXT, U_TEXT, U_TEXT, U_TEXT, U_TEXT, 4,
              U_DIST, U_DIST)
_SMALL_OFF = [None]
_acc = 0
for _r in TABLE_ROWS[1:]:
    _SMALL_OFF.append(_acc)
    _acc += D_EPART * _r
_SMALL_TOTAL = _acc

@functools.cache
def _sc_mesh():
    return plsc.VectorSubcoreMesh(core_axis_name="c", subcore_axis_name="s",
                                  num_cores=NC, num_subcores=NS)


def _sc_embed_body(*refs):
    idx_refs = refs[0:NFEAT]
    wet = refs[NFEAT]
    smalls = refs[NFEAT + 1]
    out = refs[NFEAT + 2]
    (ibuf, tflat, wecols, pbuf,
     isem, tsem, dsem, osem) = refs[NFEAT + 3:]
    c = lax.axis_index("c")
    s = lax.axis_index("s")
    base = (c * NS + s) * NODES_PER_W
    iota = lax.iota(jnp.int32, L)

    idescs = [pltpu.async_copy(idx_refs[f].at[pl.ds(base, NODES_PER_W)],
                               ibuf.at[f], isem) for f in range(NFEAT)]
    tdesc = pltpu.async_copy(smalls, tflat, tsem)
    for d in idescs:
        d.wait()

    wedescs = []
    for g in range(NODES_PER_W // L):
        v = ibuf[0, pl.ds(g * L, L)]
        v = jnp.minimum(jnp.maximum(v, 0), VOCAB - 1)
        for e in range(L):
            n = g * L + e
            wedescs.append(pltpu.async_copy(
                wet.at[:, pl.ds((v[e] >> 3) * 8, 8)],
                wecols.at[n], dsem))

    tdesc.wait()

    for f in range(1, NFEAT):
        ub = TABLE_ROWS[f] - 1
        off = _SMALL_OFF[f]
        stride = ub + 1

        def gath(g, _, ub=ub, off=off, stride=stride, f=f):
            idx16 = ibuf[f, pl.ds(g * L, L)]
            idx16 = jnp.minimum(jnp.maximum(idx16, 0), ub)
            for j in range(D_EPART):
                flat = idx16 + (off + j * stride)
                pbuf[f, j, pl.ds(g * L, L)] = plsc.load_gather(tflat, [flat])
            return 0
        lax.fori_loop(0, NODES_PER_W // L, gath, 0)

    for d in wedescs:
        d.wait()

    def we_extract(g, _):
        v = ibuf[0, pl.ds(g * L, L)]
        v = jnp.minimum(jnp.maximum(v, 0), VOCAB - 1)
        cv = v & 7
        nv = jnp.full((L,), g * L, jnp.int32) + iota
        for j in range(D_EPART):
            jf = jnp.full((L,), j, jnp.int32)
            pbuf[0, j, pl.ds(g * L, L)] = plsc.load_gather(
                wecols, [nv, jf, cv])
        return 0
    lax.fori_loop(0, NODES_PER_W // L, we_extract, 0)

    pltpu.async_copy(pbuf, out.at[:, :, pl.ds(base, NODES_PER_W)],
                     osem).wait()


@functools.cache
def _sc_embed_kernel():
    return pl.kernel(
        _sc_embed_body,
        out_type=jax.ShapeDtypeStruct((NFEAT, D_EPART, N), jnp.float32),
        mesh=_sc_mesh(),
        scratch_types=[
            pltpu.VMEM((NFEAT, NODES_PER_W), jnp.int32),
            pltpu.VMEM((_SMALL_TOTAL,), jnp.float32),
            pltpu.VMEM((NODES_PER_W, D_EPART, 8), jnp.float32),
            pltpu.VMEM((NFEAT, D_EPART, NODES_PER_W), jnp.float32),
            pltpu.SemaphoreType.DMA,
            pltpu.SemaphoreType.DMA,
            pltpu.SemaphoreType.DMA,
            pltpu.SemaphoreType.DMA,
        ],
        compiler_params=pltpu.CompilerParams(needs_layout_passes=False,
                                             use_tc_tiling_on_sc=False),
    )


def _sc_embed(*args):
    return _sc_embed_kernel()(*args)


def _sc_edge_body(h0, srcr, dstr, ewr, out, h0s, acc, s0, s1, s2, s3,
                  d0, d1, d2, d3, wbuf, msg, obuf, isem, g0sem, g1sem,
                  s0sem, s1sem):
    sidx = (s0, s1, s2, s3)
    didx = (d0, d1, d2, d3)
    gsem = (g0sem, g1sem)
    ssem = (s0sem, s1sem)
    c = lax.axis_index("c")
    s = lax.axis_index("s")
    rows = pl.ds(s * ACC_ROWS_PER_TILE, ACC_ROWS_PER_TILE)

    base = (c * NS + s) * EDGES_PER_W
    idescs = []
    for ch in range(NECH):
        off = base + ch * ECHUNK
        idescs.append(pltpu.async_copy(srcr.at[pl.ds(off, ECHUNK)],
                                       sidx[ch], isem))
        idescs.append(pltpu.async_copy(dstr.at[pl.ds(off, ECHUNK)],
                                       didx[ch], isem))
        idescs.append(pltpu.async_copy(ewr.at[pl.ds(off, ECHUNK)],
                                       wbuf.at[ch], isem))

    def zrow(i, _):
        for j in range(D_MODEL // L):
            obuf[i, pl.ds(j * L, L)] = jnp.zeros((L,), jnp.float32)
        return 0
    lax.fori_loop(0, ACC_ROWS_PER_TILE, zrow, 0)
    pltpu.sync_copy(obuf, acc.at[rows])
    pltpu.sync_copy(h0.at[rows], obuf)
    pltpu.sync_copy(obuf, h0s.at[rows])
    for d in idescs:
        d.wait()
    plsc.subcore_barrier()

    gd = [None, None]
    sd = [None, None]
    gd[0] = pltpu.async_copy(h0s.at[sidx[0]], msg.at[0], gsem[0])
    for ch in range(NECH):
        nxt = ch + 1
        if nxt < NECH:
            if nxt >= 2:
                sd[nxt % 2].wait()
            gd[nxt % 2] = pltpu.async_copy(h0s.at[sidx[nxt]],
                                           msg.at[nxt % 2], gsem[nxt % 2])
        gd[ch % 2].wait()

        def scale_group(g, _, ch=ch):
            w16 = wbuf[ch, pl.ds(g * L, L)]
            for e in range(L):
                wv = jnp.full((L,), w16[e], jnp.float32)
                i = g * L + e
                for j in range(D_MODEL // L):
                    msg[ch % 2, i, pl.ds(j * L, L)] = (
                        msg[ch % 2, i, pl.ds(j * L, L)] * wv)
            return 0
        lax.fori_loop(0, ECHUNK // L, scale_group, 0)
        sd[ch % 2] = pltpu.async_copy(msg.at[ch % 2], acc.at[didx[ch]],
                                      ssem[ch % 2], add=True)
    sd[(NECH - 2) % 2].wait()
    sd[(NECH - 1) % 2].wait()
    plsc.subcore_barrier()
    pltpu.sync_copy(acc.at[rows], obuf)
    pltpu.sync_copy(obuf, out.at[c, rows])


@functools.cache
def _sc_edges_kernel():
    return pl.kernel(
        _sc_edge_body,
        out_type=jax.ShapeDtypeStruct((NC, N, D_MODEL), jnp.float32),
        mesh=_sc_mesh(),
        scratch_types=[
            pltpu.VMEM_SHARED((N, D_MODEL), jnp.float32),
            pltpu.VMEM_SHARED((N, D_MODEL), jnp.float32),
            *[pltpu.VMEM((ECHUNK,), jnp.int32) for _ in range(2 * NECH)],
            pltpu.VMEM((NECH, ECHUNK), jnp.float32),
            pltpu.VMEM((2, ECHUNK, D_MODEL), jnp.float32),
            pltpu.VMEM((ACC_ROWS_PER_TILE, D_MODEL), jnp.float32),
            pltpu.SemaphoreType.DMA,
            pltpu.SemaphoreType.DMA,
            pltpu.SemaphoreType.DMA,
            pltpu.SemaphoreType.DMA,
            pltpu.SemaphoreType.DMA,
        ],
    )


def _sc_edges(*args):
    return _sc_edges_kernel()(*args)


def _tc_h0_body(parts_ref, w_ref, b_ref, out_ref):
    acc = None
    for f in range(NFEAT):
        t = lax.dot_general(parts_ref[f], w_ref[f],
                            dimension_numbers=(((0,), (0,)), ((), ())),
                            preferred_element_type=jnp.float32)
        acc = t if acc is None else acc + t
    out_ref[...] = acc + b_ref[...]


def _tc_h0(parts, w3, b2):
    return pl.pallas_call(
        _tc_h0_body,
        out_shape=jax.ShapeDtypeStruct((N, D_MODEL), jnp.float32),
    )(parts, w3, b2)


def _tc_prep_body(aggp, h0, fields, Wg, Wslf, bg, Wp, bp,
                  Wsh, bsh, Wst, bst, Ws0, Ws1,
                  Wgh, bgh, Wgt, bgt, Wg0, Wg1,
                  hcat_out, t_out):
    f32 = jnp.float32
    agg = aggp[0] + aggp[1]
    pre = (jnp.dot(agg, Wg[...], preferred_element_type=f32)
           + jnp.dot(h0[...], Wslf[...], preferred_element_type=f32)
           + bg[...])
    h = jnp.maximum(pre, 0.0)
    enc = jnp.dot(h, Wp[...], preferred_element_type=f32) + bp[...]
    combos = ((Wsh, bsh, Wst, bst, Ws0, Ws1),
              (Wgh, bgh, Wgt, bgt, Wg0, Wg1))
    for a, (Wh, bh, Wt, bt, W0, W1) in enumerate(combos):
        head = jnp.dot(enc, Wh[...], preferred_element_type=f32) + bh[...]
        tail = jnp.dot(enc, Wt[...], preferred_element_type=f32) + bt[...]
        hcat_out[a] = jnp.concatenate([fields[a], head], axis=0)
        t_out[a, 0] = jnp.dot(tail, W0[...], preferred_element_type=f32)
        t_out[a, 1] = jnp.dot(tail, W1[...], preferred_element_type=f32)


def _tc_prep(aggp, h0, fields, *ws):
    return pl.pallas_call(
        _tc_prep_body,
        out_shape=[
            jax.ShapeDtypeStruct((2, N + N_FIELDS, D_MODEL), jnp.float32),
            jax.ShapeDtypeStruct((2, 2, N, D_MODEL), jnp.float32),
        ],
    )(aggp, h0, fields, *ws)


_RB = 344


def _tc_score_body(h_ref, t_ref, o_ref):
    h = h_ref[0]
    s0 = lax.dot_general(h, t_ref[0, 0],
                         dimension_numbers=(((1,), (1,)), ((), ())),
                         preferred_element_type=jnp.float32)
    s1 = lax.dot_general(h, t_ref[0, 1],
                         dimension_numbers=(((1,), (1,)), ((), ())),
                         preferred_element_type=jnp.float32)
    pieces = []
    for c1 in range(N // 128):
        pieces.append(s0[:, c1 * 128:(c1 + 1) * 128])
        pieces.append(s1[:, c1 * 128:(c1 + 1) * 128])
    p = jnp.stack(pieces, axis=0)
    o_ref[0] = p.transpose(1, 0, 2)


def _tc_scores(hcat, t):
    nrows = N + N_FIELDS
    nrt = (nrows + _RB - 1) // _RB
    out4 = pl.pallas_call(
        _tc_score_body,
        grid=(2, nrt),
        in_specs=[
            pl.BlockSpec((1, _RB, D_MODEL), lambda a, r: (a, r, 0)),
            pl.BlockSpec((1, 2, N, D_MODEL), lambda a, r: (a, 0, 0, 0)),
        ],
        out_specs=pl.BlockSpec((1, _RB, 2 * (N // 128), 128),
                               lambda a, r: (a, r, 0, 0)),
        out_shape=jax.ShapeDtypeStruct((2, nrows, 2 * (N // 128), 128),
                                       jnp.float32),
    )(hcat, t)
    res = out4.reshape(2, nrows, N // 128, 2, 128)
    res = res.transpose(0, 3, 1, 2, 4)
    return res.reshape(2, 2, nrows, N)


def kernel(input_ids, token_types, n_lower, n_upper, n_alpha, n_spaces,
           n_numeric, n_special, rx_ids, ry_ids, edge_index, edge_weights,
           we_table, nl_table, nu_table, na_table, nsp_table, nnum_table,
           nspec_table, tt_table, rx_table, ry_table, W_in, b_in, W_gcn,
           W_self, b_gcn, W_proj, b_proj, Ws_head, bs_head, Ws_tail, bs_tail,
           fields_s, Ws0, Ws1, Wg_head, bg_head, Wg_tail, bg_tail, fields_g,
           Wg0, Wg1):
    i32 = jnp.int32
    idxs = [x.astype(i32) for x in
            (input_ids, n_lower, n_upper, n_alpha, n_spaces, n_numeric,
             n_special, token_types, rx_ids, ry_ids)]
    small_tables = (nl_table, nu_table, na_table, nsp_table, nnum_table,
                    nspec_table, tt_table, rx_table, ry_table)
    smalls = jnp.concatenate([t.T.reshape(-1) for t in small_tables])
    parts = _sc_embed(*idxs, we_table.T, smalls)

    w3 = W_in.reshape(NFEAT, D_EPART, D_MODEL)
    h0 = _tc_h0(parts, w3, b_in.reshape(1, D_MODEL))

    src = edge_index[0].astype(i32)
    dst = edge_index[1].astype(i32)
    aggp = _sc_edges(h0, src, dst, edge_weights)

    fields = jnp.stack([fields_s, fields_g], axis=0)
    hcat, t = _tc_prep(
        aggp, h0, fields, W_gcn, W_self, b_gcn.reshape(1, D_MODEL), W_proj,
        b_proj.reshape(1, D_MODEL),
        Ws_head, bs_head.reshape(1, D_MODEL), Ws_tail,
        bs_tail.reshape(1, D_MODEL), Ws0, Ws1,
        Wg_head, bg_head.reshape(1, D_MODEL), Wg_tail,
        bg_tail.reshape(1, D_MODEL), Wg0, Wg1)

    return _tc_scores(hcat, t)
